# Initial kernel scaffold; baseline (speedup 1.0000x reference)
import jax
import jax.numpy as jnp
from jax.experimental import pallas as pl


def kernel(x, s, edge1_index, edge2_index, batch, W1, b1, Wsc1, bsc1, W2, b2, Wsc2, bsc2, lin1_W, lin1_b, lin3_W, lin3_b):
    raise NotImplementedError("write your pallas kernel here")


# speedup vs baseline: 12.0386x; 12.0386x over previous
import functools

import jax
import jax.numpy as jnp
from jax import lax
from jax.experimental import pallas as pl
from jax.experimental.pallas import tpu as pltpu
from jax.experimental.pallas import tpu_sc as plsc

N = 10000
D = 128
NH = 128
E = 320000
NCLS = 10
KTOP = 5000

NCORES = 2
NSUB = 16
NW = NCORES * NSUB
CHUNK = 128
CPW = 80
EPAD = NW * CPW * CHUNK
ECH = EPAD // CHUNK
NROW = 10240
DUMMY = 10016
RPS = NROW // NSUB

_HI = lax.Precision.HIGHEST


def _mesh():
    return plsc.VectorSubcoreMesh(core_axis_name="c", subcore_axis_name="s")


_SC_PARAMS = pltpu.CompilerParams(use_tc_tiling_on_sc=False,
                                  needs_layout_passes=False)


def _sc_degrees(dst1r, dst2r):
    @functools.partial(
        pl.kernel,
        out_type=jax.ShapeDtypeStruct((NW, 2, NROW), jnp.float32),
        mesh=_mesh(),
        compiler_params=_SC_PARAMS,
        scratch_types=[
            pltpu.VMEM((CPW, CHUNK), jnp.int32),
            pltpu.VMEM((CPW, CHUNK), jnp.int32),
            pltpu.VMEM((NROW,), jnp.float32),
            pltpu.VMEM((NROW,), jnp.float32),
        ],
    )
    def deg_kernel(d1_hbm, d2_hbm, out_hbm, d1_v, d2_v, a1_v, a2_v):
        cid = lax.axis_index("c")
        sid = lax.axis_index("s")
        wid = sid * NCORES + cid
        base = wid * CPW
        pltpu.sync_copy(d1_hbm.at[pl.ds(base, CPW)], d1_v)
        pltpu.sync_copy(d2_hbm.at[pl.ds(base, CPW)], d2_v)
        z16 = jnp.zeros((16,), jnp.float32)
        ones = jnp.ones((16,), jnp.float32)

        def zbody(i, carry):
            a1_v[pl.ds(i * 16, 16)] = z16
            a2_v[pl.ds(i * 16, 16)] = z16
            return carry

        lax.fori_loop(0, NROW // 16, zbody, 0)

        def ebody(i, carry):
            r = i // (CHUNK // 16)
            c = (i % (CHUNK // 16)) * 16
            plsc.addupdate_scatter(a1_v, [d1_v[r, pl.ds(c, 16)]], ones)
            plsc.addupdate_scatter(a2_v, [d2_v[r, pl.ds(c, 16)]], ones)
            return carry

        lax.fori_loop(0, CPW * (CHUNK // 16), ebody, 0)
        pltpu.sync_copy(a1_v, out_hbm.at[wid, 0])
        pltpu.sync_copy(a2_v, out_hbm.at[wid, 1])

    return deg_kernel(dst1r, dst2r)


def _sc_row_agg(hs, srcr, dstr):
    @functools.partial(
        pl.kernel,
        out_type=jax.ShapeDtypeStruct((NCORES, NROW, NH), jnp.float32),
        mesh=_mesh(),
        compiler_params=_SC_PARAMS,
        scratch_types=[
            pltpu.VMEM((CPW, CHUNK), jnp.int32),
            pltpu.VMEM((CPW, CHUNK), jnp.int32),
            pltpu.VMEM((CHUNK, NH), jnp.float32),
            pltpu.VMEM((16, NH), jnp.float32),
            pltpu.VMEM_SHARED((NROW, NH), jnp.float32),
            pltpu.SemaphoreType.DMA,
        ],
    )
    def rowagg_kernel(hs_hbm, src_hbm, dst_hbm, out_hbm,
                      src_v, dst_v, rows_v, zb_v, acc_sh, sem):
        cid = lax.axis_index("c")
        sid = lax.axis_index("s")
        wid = sid * NCORES + cid
        base = wid * CPW
        pltpu.sync_copy(src_hbm.at[pl.ds(base, CPW)], src_v)
        pltpu.sync_copy(dst_hbm.at[pl.ds(base, CPW)], dst_v)
        z16 = jnp.zeros((16,), jnp.float32)

        def zb_body(i, carry):
            zb_v[i // 8, pl.ds((i % 8) * 16, 16)] = z16
            return carry

        lax.fori_loop(0, 16 * (NH // 16), zb_body, 0)
        row0 = sid * RPS

        def zacc_body(t, carry):
            pltpu.sync_copy(zb_v, acc_sh.at[pl.ds(row0 + t * 16, 16)])
            return carry

        lax.fori_loop(0, RPS // 16, zacc_body, 0)
        plsc.subcore_barrier()

        def ebody(j, carry):
            pltpu.async_copy(hs_hbm.at[src_v.at[j]], rows_v, sem).wait()
            pltpu.sync_copy(rows_v, acc_sh.at[dst_v.at[j]], add=True)
            return carry

        lax.fori_loop(0, CPW, ebody, 0)
        plsc.subcore_barrier()
        pltpu.sync_copy(acc_sh.at[pl.ds(row0, RPS)],
                        out_hbm.at[cid, pl.ds(row0, RPS)])

    return rowagg_kernel(hs, srcr, dstr)


def _sc_scalar_agg(tab, srcr, dstr):
    @functools.partial(
        pl.kernel,
        out_type=jax.ShapeDtypeStruct((NW, NROW), jnp.float32),
        mesh=_mesh(),
        compiler_params=_SC_PARAMS,
        scratch_types=[
            pltpu.VMEM((CPW, CHUNK), jnp.int32),
            pltpu.VMEM((CPW, CHUNK), jnp.int32),
            pltpu.VMEM((NROW,), jnp.float32),
            pltpu.VMEM((NROW,), jnp.float32),
        ],
    )
    def scal_kernel(tab_hbm, src_hbm, dst_hbm, out_hbm,
                    src_v, dst_v, tab_v, acc_v):
        cid = lax.axis_index("c")
        sid = lax.axis_index("s")
        wid = sid * NCORES + cid
        base = wid * CPW
        pltpu.sync_copy(src_hbm.at[pl.ds(base, CPW)], src_v)
        pltpu.sync_copy(dst_hbm.at[pl.ds(base, CPW)], dst_v)
        pltpu.sync_copy(tab_hbm, tab_v)
        z16 = jnp.zeros((16,), jnp.float32)

        def zbody(i, carry):
            acc_v[pl.ds(i * 16, 16)] = z16
            return carry

        lax.fori_loop(0, NROW // 16, zbody, 0)

        def ebody(i, carry):
            r = i // (CHUNK // 16)
            c = (i % (CHUNK // 16)) * 16
            vals = plsc.load_gather(tab_v, [src_v[r, pl.ds(c, 16)]])
            plsc.addupdate_scatter(acc_v, [dst_v[r, pl.ds(c, 16)]], vals)
            return carry

        lax.fori_loop(0, CPW * (CHUNK // 16), ebody, 0)
        pltpu.sync_copy(acc_v, out_hbm.at[wid])

    return scal_kernel(tab, srcr, dstr)


def _tc_prep(degp, x, w1):
    def body(degp_ref, x_ref, w1_ref, d1_ref, d2_ref, hs1_ref):
        deg = jnp.sum(degp_ref[...], axis=0) + 1.0
        dinv = lax.rsqrt(deg)
        d1 = dinv[0]
        d2 = dinv[1]
        d1_ref[...] = d1
        d2_ref[...] = d2
        h0 = jnp.dot(x_ref[...], w1_ref[...],
                     preferred_element_type=jnp.float32, precision=_HI)
        hs1_ref[...] = h0 * d1[0:N][:, None]

    return pl.pallas_call(
        body,
        out_shape=(
            jax.ShapeDtypeStruct((NROW,), jnp.float32),
            jax.ShapeDtypeStruct((NROW,), jnp.float32),
            jax.ShapeDtypeStruct((N, NH), jnp.float32),
        ),
    )(degp, x, w1)


def _tc_big_matmul(s, w2, dinv2col):
    MB = 200
    nm = N // MB

    def body(s_ref, w_ref, d_ref, o_ref):
        o_ref[...] = jnp.dot(s_ref[...], w_ref[...],
                             preferred_element_type=jnp.float32,
                             precision=_HI) * d_ref[...]

    return pl.pallas_call(
        body,
        grid=(nm,),
        in_specs=[
            pl.BlockSpec((MB, N), lambda i: (i, 0)),
            pl.BlockSpec((N, NH), lambda i: (0, 0)),
            pl.BlockSpec((MB, 1), lambda i: (i, 0)),
        ],
        out_specs=pl.BlockSpec((MB, NH), lambda i: (i, 0)),
        out_shape=jax.ShapeDtypeStruct((N, NH), jnp.float32),
    )(s, w2, dinv2col)


def _tc_combine(aggp, hs, dinv, b, wsc_row):
    def body(aggp_ref, hs_ref, d_ref, b_ref, wsc_ref, h_ref, ps_ref):
        a = aggp_ref[...]
        agg = a[0] + a[1]
        d = d_ref[...]
        base = agg[0:N] + hs_ref[...]
        hval = jnp.maximum(base * d[0:N][:, None] + b_ref[...][None, :], 0.0)
        hfull = jnp.concatenate(
            [hval, jnp.zeros((NROW - N, NH), jnp.float32)], axis=0)
        h_ref[...] = hfull
        p = jnp.sum(hfull * wsc_ref[...], axis=1)
        ps_ref[...] = d * p

    return pl.pallas_call(
        body,
        out_shape=(
            jax.ShapeDtypeStruct((NROW, NH), jnp.float32),
            jax.ShapeDtypeStruct((NROW,), jnp.float32),
        ),
    )(aggp, hs, dinv, b, wsc_row)


def _tc_readout(sagp, dinv, ps, bsc, h):
    def body(sagp_ref, d_ref, ps_ref, bsc_ref, h_ref, out_ref):
        sagg = jnp.sum(sagp_ref[...], axis=0)
        d = d_ref[...]
        score = d * (sagg + ps_ref[...]) + bsc_ref[0]

        bits = lax.bitcast_convert_type(score, jnp.uint32)
        key = jnp.where(bits >> 31 != 0, ~bits,
                        bits | jnp.uint32(0x80000000))
        idx = lax.broadcasted_iota(jnp.int32, (NROW,), 0)
        key = jnp.where(idx < N, key, jnp.uint32(0))

        t = jnp.uint32(0)
        for bit in range(31, -1, -1):
            cand = t | jnp.uint32(1 << bit)
            cnt = jnp.sum((key >= cand).astype(jnp.int32))
            t = jnp.where(cnt >= KTOP, cand, t)
        c_gt = jnp.sum((key > t).astype(jnp.int32))

        ties = (key == t)
        jcut = jnp.int32(0)
        for bit in range(13, -1, -1):
            cand = jcut + jnp.int32(1 << bit)
            f = c_gt + jnp.sum((ties & (idx < cand)).astype(jnp.int32))
            jcut = jnp.where(f < KTOP, cand, jcut)
        mask = (key > t) | (ties & (idx < jcut + 1))

        g = jnp.tanh(score)
        gated = h_ref[...] * g[:, None]
        mcol = mask.astype(jnp.float32)[:, None]
        xmax = jnp.max(jnp.where(mcol > 0.0, gated, -jnp.inf), axis=0)
        xsum = jnp.sum(gated * mcol, axis=0)
        out_ref[0, :] = xmax
        out_ref[1, :] = xsum / KTOP

    return pl.pallas_call(
        body,
        out_shape=jax.ShapeDtypeStruct((2, NH), jnp.float32),
    )(sagp, dinv, ps, bsc, h)


def _tc_head(x1p, x2p, lin1w, lin1b, lin3w, lin3b):
    def body(x1_ref, x2_ref, w1_ref, b1_ref, w3_ref, b3_ref, out_ref):
        z = jnp.concatenate(
            [x1_ref[0], x1_ref[1], x2_ref[0], x2_ref[1]], axis=0)
        zm = jnp.sum(z[:, None] * w1_ref[...], axis=0) + b1_ref[...]
        zm = jnp.maximum(zm, 0.0)
        o = jnp.sum(zm[:, None] * w3_ref[...], axis=0) + b3_ref[...]
        m = jnp.max(o)
        e = o - m
        out_ref[...] = (e - jnp.log(jnp.sum(jnp.exp(e))))[None, :]

    return pl.pallas_call(
        body,
        out_shape=jax.ShapeDtypeStruct((1, NCLS), jnp.float32),
    )(x1p, x2p, lin1w, lin1b, lin3w, lin3b)


def _pad_edges(ei):
    pad = EPAD - E
    srcp = jnp.concatenate(
        [ei[0], jnp.zeros((pad,), jnp.int32)]).reshape(ECH, CHUNK)
    dstp = jnp.concatenate(
        [ei[1], jnp.full((pad,), DUMMY, jnp.int32)]).reshape(ECH, CHUNK)
    return srcp, dstp


def kernel(x, s, edge1_index, edge2_index, batch, W1, b1, Wsc1, bsc1,
           W2, b2, Wsc2, bsc2, lin1_W, lin1_b, lin3_W, lin3_b):
    src1r, dst1r = _pad_edges(edge1_index)
    src2r, dst2r = _pad_edges(edge2_index)

    degp = _sc_degrees(dst1r, dst2r)
    dinv1, dinv2, hs1 = _tc_prep(degp, x, W1)
    hs2 = _tc_big_matmul(s, W2, dinv2.reshape(NROW, 1))

    aggp1 = _sc_row_agg(hs1, src1r, dst1r)
    h1, ps1 = _tc_combine(aggp1, hs1, dinv1, b1, Wsc1.reshape(1, NH))
    sagp1 = _sc_scalar_agg(ps1, src1r, dst1r)
    x1p = _tc_readout(sagp1, dinv1, ps1, bsc1, h1)

    aggp2 = _sc_row_agg(hs2, src2r, dst2r)
    h2, ps2 = _tc_combine(aggp2, hs2, dinv2, b2, Wsc2.reshape(1, NH))
    sagp2 = _sc_scalar_agg(ps2, src2r, dst2r)
    x2p = _tc_readout(sagp2, dinv2, ps2, bsc2, h2)

    return _tc_head(x1p, x2p, lin1_W, lin1_b, lin3_W, lin3_b)


# speedup vs baseline: 13.2470x; 1.1004x over previous
import functools

import jax
import jax.numpy as jnp
from jax import lax
from jax.experimental import pallas as pl
from jax.experimental.pallas import tpu as pltpu
from jax.experimental.pallas import tpu_sc as plsc

N = 10000
D = 128
NH = 128
E = 320000
NCLS = 10
KTOP = 5000

NCORES = 2
NSUB = 16
NW = NCORES * NSUB
CHUNK = 128
CPW = 80
EPAD = NW * CPW * CHUNK
ECH = EPAD // CHUNK
NROW = 10240
DUMMY = 10016
RPS = NROW // NSUB
NBUF = 2

_HI = lax.Precision.HIGHEST


def _mesh():
    return plsc.VectorSubcoreMesh(core_axis_name="c", subcore_axis_name="s")


_SC_PARAMS = pltpu.CompilerParams(use_tc_tiling_on_sc=False,
                                  needs_layout_passes=False)


def _sc_degrees(dst1r, dst2r):
    @functools.partial(
        pl.kernel,
        out_type=jax.ShapeDtypeStruct((NW, 2, NROW), jnp.float32),
        mesh=_mesh(),
        compiler_params=_SC_PARAMS,
        scratch_types=[
            pltpu.VMEM((CPW, CHUNK), jnp.int32),
            pltpu.VMEM((CPW, CHUNK), jnp.int32),
            pltpu.VMEM((NROW,), jnp.float32),
            pltpu.VMEM((NROW,), jnp.float32),
        ],
    )
    def deg_kernel(d1_hbm, d2_hbm, out_hbm, d1_v, d2_v, a1_v, a2_v):
        cid = lax.axis_index("c")
        sid = lax.axis_index("s")
        wid = sid * NCORES + cid
        base = wid * CPW
        pltpu.sync_copy(d1_hbm.at[pl.ds(base, CPW)], d1_v)
        pltpu.sync_copy(d2_hbm.at[pl.ds(base, CPW)], d2_v)
        z16 = jnp.zeros((16,), jnp.float32)
        ones = jnp.ones((16,), jnp.float32)

        def zbody(i, carry):
            a1_v[pl.ds(i * 16, 16)] = z16
            a2_v[pl.ds(i * 16, 16)] = z16
            return carry

        lax.fori_loop(0, NROW // 16, zbody, 0)

        def ebody(i, carry):
            r = i // (CHUNK // 16)
            c = (i % (CHUNK // 16)) * 16
            plsc.addupdate_scatter(a1_v, [d1_v[r, pl.ds(c, 16)]], ones)
            plsc.addupdate_scatter(a2_v, [d2_v[r, pl.ds(c, 16)]], ones)
            return carry

        lax.fori_loop(0, CPW * (CHUNK // 16), ebody, 0)
        pltpu.sync_copy(a1_v, out_hbm.at[wid, 0])
        pltpu.sync_copy(a2_v, out_hbm.at[wid, 1])

    return deg_kernel(dst1r, dst2r)


def _sc_row_agg(hs, srcr, dstr):
    @functools.partial(
        pl.kernel,
        out_type=jax.ShapeDtypeStruct((NCORES, NROW, NH), jnp.float32),
        mesh=_mesh(),
        compiler_params=_SC_PARAMS,
        scratch_types=[
            pltpu.VMEM((CPW, CHUNK), jnp.int32),
            pltpu.VMEM((NBUF, CHUNK), jnp.int32),
            pltpu.VMEM((NBUF, CHUNK, NH), jnp.float32),
            pltpu.VMEM_SHARED((NROW, NH), jnp.float32),
            [pltpu.SemaphoreType.DMA] * NBUF,
            [pltpu.SemaphoreType.DMA] * NBUF,
        ],
    )
    def rowagg_kernel(hs_hbm, src_hbm, dst_hbm, out_hbm,
                      src_v, dstr_v, rows_v, acc_sh, gsems, dsems):
        cid = lax.axis_index("c")
        sid = lax.axis_index("s")
        wid = sid * NCORES + cid
        base = wid * CPW
        pltpu.sync_copy(src_hbm.at[pl.ds(base, CPW)], src_v)
        z16 = jnp.zeros((16,), jnp.float32)

        def zb_body(i, carry):
            rows_v[0, i // 8, pl.ds((i % 8) * 16, 16)] = z16
            return carry

        lax.fori_loop(0, CHUNK * (NH // 16), zb_body, 0)
        row0 = sid * RPS

        def zacc_body(t, carry):
            pltpu.sync_copy(rows_v.at[0],
                            acc_sh.at[pl.ds(row0 + t * CHUNK, CHUNK)])
            return carry

        lax.fori_loop(0, RPS // CHUNK, zacc_body, 0)
        plsc.subcore_barrier()

        def start_chunk(j, b):
            pltpu.async_copy(dst_hbm.at[base + j], dstr_v.at[b], dsems[b])
            pltpu.async_copy(hs_hbm.at[src_v.at[j]], rows_v.at[b], gsems[b])

        def wait_chunk(b):
            pltpu.make_async_copy(dst_hbm.at[0], dstr_v.at[b],
                                  dsems[b]).wait()
            pltpu.make_async_copy(hs_hbm.at[pl.ds(0, CHUNK)],
                                  rows_v.at[b], gsems[b]).wait()

        for b in range(NBUF):
            start_chunk(b, b)

        def ebody(g, carry):
            for b in range(NBUF):
                j = g * NBUF + b
                wait_chunk(b)
                pltpu.sync_copy(rows_v.at[b], acc_sh.at[dstr_v.at[b]],
                                add=True)
                jn = j + NBUF

                @pl.when(jn < CPW)
                def _():
                    start_chunk(jn, b)

            return carry

        lax.fori_loop(0, CPW // NBUF, ebody, 0)
        plsc.subcore_barrier()
        pltpu.sync_copy(acc_sh.at[pl.ds(row0, RPS)],
                        out_hbm.at[cid, pl.ds(row0, RPS)])

    return rowagg_kernel(hs, srcr, dstr)


def _sc_scalar_agg(tab, srcr, dstr):
    @functools.partial(
        pl.kernel,
        out_type=jax.ShapeDtypeStruct((NW, NROW), jnp.float32),
        mesh=_mesh(),
        compiler_params=_SC_PARAMS,
        scratch_types=[
            pltpu.VMEM((CPW, CHUNK), jnp.int32),
            pltpu.VMEM((CPW, CHUNK), jnp.int32),
            pltpu.VMEM((NROW,), jnp.float32),
            pltpu.VMEM((NROW,), jnp.float32),
        ],
    )
    def scal_kernel(tab_hbm, src_hbm, dst_hbm, out_hbm,
                    src_v, dst_v, tab_v, acc_v):
        cid = lax.axis_index("c")
        sid = lax.axis_index("s")
        wid = sid * NCORES + cid
        base = wid * CPW
        pltpu.sync_copy(src_hbm.at[pl.ds(base, CPW)], src_v)
        pltpu.sync_copy(dst_hbm.at[pl.ds(base, CPW)], dst_v)
        pltpu.sync_copy(tab_hbm, tab_v)
        z16 = jnp.zeros((16,), jnp.float32)

        def zbody(i, carry):
            acc_v[pl.ds(i * 16, 16)] = z16
            return carry

        lax.fori_loop(0, NROW // 16, zbody, 0)

        def ebody(i, carry):
            r = i // (CHUNK // 16)
            c = (i % (CHUNK // 16)) * 16
            vals = plsc.load_gather(tab_v, [src_v[r, pl.ds(c, 16)]])
            plsc.addupdate_scatter(acc_v, [dst_v[r, pl.ds(c, 16)]], vals)
            return carry

        lax.fori_loop(0, CPW * (CHUNK // 16), ebody, 0)
        pltpu.sync_copy(acc_v, out_hbm.at[wid])

    return scal_kernel(tab, srcr, dstr)


def _tc_prep(degp, x, w1):
    def body(degp_ref, x_ref, w1_ref, d1_ref, d2_ref, hs1_ref):
        deg = jnp.sum(degp_ref[...], axis=0) + 1.0
        dinv = lax.rsqrt(deg)
        d1 = dinv[0]
        d2 = dinv[1]
        d1_ref[...] = d1
        d2_ref[...] = d2
        h0 = jnp.dot(x_ref[...], w1_ref[...],
                     preferred_element_type=jnp.float32, precision=_HI)
        hs1_ref[...] = h0 * d1[0:N][:, None]

    return pl.pallas_call(
        body,
        out_shape=(
            jax.ShapeDtypeStruct((NROW,), jnp.float32),
            jax.ShapeDtypeStruct((NROW,), jnp.float32),
            jax.ShapeDtypeStruct((N, NH), jnp.float32),
        ),
    )(degp, x, w1)


def _tc_big_matmul(s, w2, dinv2col):
    MB = 200
    nm = N // MB

    def body(s_ref, w_ref, d_ref, o_ref):
        o_ref[...] = jnp.dot(s_ref[...], w_ref[...],
                             preferred_element_type=jnp.float32,
                             precision=_HI) * d_ref[...]

    return pl.pallas_call(
        body,
        grid=(nm,),
        in_specs=[
            pl.BlockSpec((MB, N), lambda i: (i, 0)),
            pl.BlockSpec((N, NH), lambda i: (0, 0)),
            pl.BlockSpec((MB, 1), lambda i: (i, 0)),
        ],
        out_specs=pl.BlockSpec((MB, NH), lambda i: (i, 0)),
        out_shape=jax.ShapeDtypeStruct((N, NH), jnp.float32),
    )(s, w2, dinv2col)


def _tc_combine(aggp, hs, dinv, b, wsc_row):
    def body(aggp_ref, hs_ref, d_ref, b_ref, wsc_ref, h_ref, ps_ref):
        a = aggp_ref[...]
        agg = a[0] + a[1]
        d = d_ref[...]
        base = agg[0:N] + hs_ref[...]
        hval = jnp.maximum(base * d[0:N][:, None] + b_ref[...][None, :], 0.0)
        hfull = jnp.concatenate(
            [hval, jnp.zeros((NROW - N, NH), jnp.float32)], axis=0)
        h_ref[...] = hfull
        p = jnp.sum(hfull * wsc_ref[...], axis=1)
        ps_ref[...] = d * p

    return pl.pallas_call(
        body,
        out_shape=(
            jax.ShapeDtypeStruct((NROW, NH), jnp.float32),
            jax.ShapeDtypeStruct((NROW,), jnp.float32),
        ),
    )(aggp, hs, dinv, b, wsc_row)


def _tc_readout(sagp, dinv, ps, bsc, h):
    def body(sagp_ref, d_ref, ps_ref, bsc_ref, h_ref, out_ref):
        sagg = jnp.sum(sagp_ref[...], axis=0)
        d = d_ref[...]
        score = d * (sagg + ps_ref[...]) + bsc_ref[0]

        bits = lax.bitcast_convert_type(score, jnp.uint32)
        key = jnp.where(bits >> 31 != 0, ~bits,
                        bits | jnp.uint32(0x80000000))
        idx = lax.broadcasted_iota(jnp.int32, (NROW,), 0)
        key = jnp.where(idx < N, key, jnp.uint32(0))

        t = jnp.uint32(0)
        for bit in range(31, -1, -1):
            cand = t | jnp.uint32(1 << bit)
            cnt = jnp.sum((key >= cand).astype(jnp.int32))
            t = jnp.where(cnt >= KTOP, cand, t)
        c_gt = jnp.sum((key > t).astype(jnp.int32))

        ties = (key == t)
        jcut = jnp.int32(0)
        for bit in range(13, -1, -1):
            cand = jcut + jnp.int32(1 << bit)
            f = c_gt + jnp.sum((ties & (idx < cand)).astype(jnp.int32))
            jcut = jnp.where(f < KTOP, cand, jcut)
        mask = (key > t) | (ties & (idx < jcut + 1))

        g = jnp.tanh(score)
        gated = h_ref[...] * g[:, None]
        mcol = mask.astype(jnp.float32)[:, None]
        xmax = jnp.max(jnp.where(mcol > 0.0, gated, -jnp.inf), axis=0)
        xsum = jnp.sum(gated * mcol, axis=0)
        out_ref[0, :] = xmax
        out_ref[1, :] = xsum / KTOP

    return pl.pallas_call(
        body,
        out_shape=jax.ShapeDtypeStruct((2, NH), jnp.float32),
    )(sagp, dinv, ps, bsc, h)


def _tc_head(x1p, x2p, lin1w, lin1b, lin3w, lin3b):
    def body(x1_ref, x2_ref, w1_ref, b1_ref, w3_ref, b3_ref, out_ref):
        z = jnp.concatenate(
            [x1_ref[0], x1_ref[1], x2_ref[0], x2_ref[1]], axis=0)
        zm = jnp.sum(z[:, None] * w1_ref[...], axis=0) + b1_ref[...]
        zm = jnp.maximum(zm, 0.0)
        o = jnp.sum(zm[:, None] * w3_ref[...], axis=0) + b3_ref[...]
        m = jnp.max(o)
        e = o - m
        out_ref[...] = (e - jnp.log(jnp.sum(jnp.exp(e))))[None, :]

    return pl.pallas_call(
        body,
        out_shape=jax.ShapeDtypeStruct((1, NCLS), jnp.float32),
    )(x1p, x2p, lin1w, lin1b, lin3w, lin3b)


def _pad_edges(ei):
    pad = EPAD - E
    srcp = jnp.concatenate(
        [ei[0], jnp.zeros((pad,), jnp.int32)]).reshape(ECH, CHUNK)
    dstp = jnp.concatenate(
        [ei[1], jnp.full((pad,), DUMMY, jnp.int32)]).reshape(ECH, CHUNK)
    return srcp, dstp


def kernel(x, s, edge1_index, edge2_index, batch, W1, b1, Wsc1, bsc1,
           W2, b2, Wsc2, bsc2, lin1_W, lin1_b, lin3_W, lin3_b):
    src1r, dst1r = _pad_edges(edge1_index)
    src2r, dst2r = _pad_edges(edge2_index)

    degp = _sc_degrees(dst1r, dst2r)
    dinv1, dinv2, hs1 = _tc_prep(degp, x, W1)
    hs2 = _tc_big_matmul(s, W2, dinv2.reshape(NROW, 1))

    aggp1 = _sc_row_agg(hs1, src1r, dst1r)
    h1, ps1 = _tc_combine(aggp1, hs1, dinv1, b1, Wsc1.reshape(1, NH))
    sagp1 = _sc_scalar_agg(ps1, src1r, dst1r)
    x1p = _tc_readout(sagp1, dinv1, ps1, bsc1, h1)

    aggp2 = _sc_row_agg(hs2, src2r, dst2r)
    h2, ps2 = _tc_combine(aggp2, hs2, dinv2, b2, Wsc2.reshape(1, NH))
    sagp2 = _sc_scalar_agg(ps2, src2r, dst2r)
    x2p = _tc_readout(sagp2, dinv2, ps2, bsc2, h2)

    return _tc_head(x1p, x2p, lin1_W, lin1_b, lin3_W, lin3_b)


# speedup vs baseline: 15.5124x; 1.1710x over previous
import functools

import jax
import jax.numpy as jnp
from jax import lax
from jax.experimental import pallas as pl
from jax.experimental.pallas import tpu as pltpu
from jax.experimental.pallas import tpu_sc as plsc

N = 10000
D = 128
NH = 128
E = 320000
NCLS = 10
KTOP = 5000

NCORES = 2
NSUB = 16
NW = NCORES * NSUB
CHUNK = 128
CPW = 80
EPAD = NW * CPW * CHUNK
ECH = EPAD // CHUNK
NROW = 10240
DUMMY = 10016
RPS = NROW // NSUB
NBUF = 2

_HI = lax.Precision.HIGHEST


def _mesh():
    return plsc.VectorSubcoreMesh(core_axis_name="c", subcore_axis_name="s")


_SC_PARAMS = pltpu.CompilerParams(use_tc_tiling_on_sc=False,
                                  needs_layout_passes=False)


def _sc_degrees(dst1r, dst2r):
    @functools.partial(
        pl.kernel,
        out_type=jax.ShapeDtypeStruct((NW, 2, NROW), jnp.float32),
        mesh=_mesh(),
        compiler_params=_SC_PARAMS,
        scratch_types=[
            pltpu.VMEM((CPW, CHUNK), jnp.int32),
            pltpu.VMEM((CPW, CHUNK), jnp.int32),
            pltpu.VMEM((NROW,), jnp.float32),
            pltpu.VMEM((NROW,), jnp.float32),
        ],
    )
    def deg_kernel(d1_hbm, d2_hbm, out_hbm, d1_v, d2_v, a1_v, a2_v):
        cid = lax.axis_index("c")
        sid = lax.axis_index("s")
        wid = sid * NCORES + cid
        base = wid * CPW
        pltpu.sync_copy(d1_hbm.at[pl.ds(base, CPW)], d1_v)
        pltpu.sync_copy(d2_hbm.at[pl.ds(base, CPW)], d2_v)
        z16 = jnp.zeros((16,), jnp.float32)
        ones = jnp.ones((16,), jnp.float32)

        def zbody(i, carry):
            a1_v[pl.ds(i * 16, 16)] = z16
            a2_v[pl.ds(i * 16, 16)] = z16
            return carry

        lax.fori_loop(0, NROW // 16, zbody, 0)

        def ebody(i, carry):
            r = i // (CHUNK // 16)
            c = (i % (CHUNK // 16)) * 16
            plsc.addupdate_scatter(a1_v, [d1_v[r, pl.ds(c, 16)]], ones)
            plsc.addupdate_scatter(a2_v, [d2_v[r, pl.ds(c, 16)]], ones)
            return carry

        lax.fori_loop(0, CPW * (CHUNK // 16), ebody, 0)
        pltpu.sync_copy(a1_v, out_hbm.at[wid, 0])
        pltpu.sync_copy(a2_v, out_hbm.at[wid, 1])

    return deg_kernel(dst1r, dst2r)


def _sc_row_agg(hs, srcr, dstr):
    @functools.partial(
        pl.kernel,
        out_type=jax.ShapeDtypeStruct((NCORES, NROW, NH), jnp.float32),
        mesh=_mesh(),
        compiler_params=_SC_PARAMS,
        scratch_types=[
            pltpu.VMEM((CPW, CHUNK), jnp.int32),
            pltpu.VMEM((NBUF, CHUNK), jnp.int32),
            pltpu.VMEM((NBUF, CHUNK, NH), jnp.float32),
            pltpu.VMEM_SHARED((NROW, NH), jnp.float32),
            [pltpu.SemaphoreType.DMA] * NBUF,
            [pltpu.SemaphoreType.DMA] * NBUF,
        ],
    )
    def rowagg_kernel(hs_hbm, src_hbm, dst_hbm, out_hbm,
                      src_v, dstr_v, rows_v, acc_sh, gsems, dsems):
        cid = lax.axis_index("c")
        sid = lax.axis_index("s")
        wid = sid * NCORES + cid
        base = wid * CPW
        pltpu.sync_copy(src_hbm.at[pl.ds(base, CPW)], src_v)
        z16 = jnp.zeros((16,), jnp.float32)

        def zb_body(i, carry):
            rows_v[0, i // 8, pl.ds((i % 8) * 16, 16)] = z16
            return carry

        lax.fori_loop(0, CHUNK * (NH // 16), zb_body, 0)
        row0 = sid * RPS

        def zacc_body(t, carry):
            pltpu.sync_copy(rows_v.at[0],
                            acc_sh.at[pl.ds(row0 + t * CHUNK, CHUNK)])
            return carry

        lax.fori_loop(0, RPS // CHUNK, zacc_body, 0)
        plsc.subcore_barrier()

        def start_chunk(j, b):
            pltpu.async_copy(dst_hbm.at[base + j], dstr_v.at[b], dsems[b])
            pltpu.async_copy(hs_hbm.at[src_v.at[j]], rows_v.at[b], gsems[b])

        def wait_chunk(b):
            pltpu.make_async_copy(dst_hbm.at[0], dstr_v.at[b],
                                  dsems[b]).wait()
            pltpu.make_async_copy(hs_hbm.at[pl.ds(0, CHUNK)],
                                  rows_v.at[b], gsems[b]).wait()

        for b in range(NBUF):
            start_chunk(b, b)

        def ebody(g, carry):
            for b in range(NBUF):
                j = g * NBUF + b
                wait_chunk(b)
                pltpu.sync_copy(rows_v.at[b], acc_sh.at[dstr_v.at[b]],
                                add=True)
                jn = j + NBUF

                @pl.when(jn < CPW)
                def _():
                    start_chunk(jn, b)

            return carry

        lax.fori_loop(0, CPW // NBUF, ebody, 0)
        plsc.subcore_barrier()
        pltpu.sync_copy(acc_sh.at[pl.ds(row0, RPS)],
                        out_hbm.at[cid, pl.ds(row0, RPS)])

    return rowagg_kernel(hs, srcr, dstr)


def _sc_scalar_agg(tab, srcr, dstr):
    @functools.partial(
        pl.kernel,
        out_type=jax.ShapeDtypeStruct((NW, NROW), jnp.float32),
        mesh=_mesh(),
        compiler_params=_SC_PARAMS,
        scratch_types=[
            pltpu.VMEM((CPW, CHUNK), jnp.int32),
            pltpu.VMEM((CPW, CHUNK), jnp.int32),
            pltpu.VMEM((NROW,), jnp.float32),
            pltpu.VMEM((NROW,), jnp.float32),
        ],
    )
    def scal_kernel(tab_hbm, src_hbm, dst_hbm, out_hbm,
                    src_v, dst_v, tab_v, acc_v):
        cid = lax.axis_index("c")
        sid = lax.axis_index("s")
        wid = sid * NCORES + cid
        base = wid * CPW
        pltpu.sync_copy(src_hbm.at[pl.ds(base, CPW)], src_v)
        pltpu.sync_copy(dst_hbm.at[pl.ds(base, CPW)], dst_v)
        pltpu.sync_copy(tab_hbm, tab_v)
        z16 = jnp.zeros((16,), jnp.float32)

        def zbody(i, carry):
            acc_v[pl.ds(i * 16, 16)] = z16
            return carry

        lax.fori_loop(0, NROW // 16, zbody, 0)

        def ebody(i, carry):
            r = i // (CHUNK // 16)
            c = (i % (CHUNK // 16)) * 16
            vals = plsc.load_gather(tab_v, [src_v[r, pl.ds(c, 16)]])
            plsc.addupdate_scatter(acc_v, [dst_v[r, pl.ds(c, 16)]], vals)
            return carry

        lax.fori_loop(0, CPW * (CHUNK // 16), ebody, 0)
        pltpu.sync_copy(acc_v, out_hbm.at[wid])

    return scal_kernel(tab, srcr, dstr)


def _tc_prep(degp, x, w1):
    def body(degp_ref, x_ref, w1_ref, d1_ref, d2_ref, hs1_ref):
        deg = jnp.sum(degp_ref[...], axis=0) + 1.0
        dinv = lax.rsqrt(deg)
        d1 = dinv[0]
        d2 = dinv[1]
        d1_ref[...] = d1
        d2_ref[...] = d2
        h0 = jnp.dot(x_ref[...], w1_ref[...],
                     preferred_element_type=jnp.float32, precision=_HI)
        hs1_ref[...] = h0 * d1[0:N][:, None]

    return pl.pallas_call(
        body,
        out_shape=(
            jax.ShapeDtypeStruct((NROW,), jnp.float32),
            jax.ShapeDtypeStruct((NROW,), jnp.float32),
            jax.ShapeDtypeStruct((N, NH), jnp.float32),
        ),
    )(degp, x, w1)


def _tc_big_matmul(s, w2, dinv2col):
    MB = 200
    nm = N // MB

    def body(s_ref, w_ref, d_ref, o_ref):
        o_ref[...] = jnp.dot(s_ref[...], w_ref[...],
                             preferred_element_type=jnp.float32,
                             precision=_HI) * d_ref[...]

    return pl.pallas_call(
        body,
        grid=(nm,),
        in_specs=[
            pl.BlockSpec((MB, N), lambda i: (i, 0)),
            pl.BlockSpec((N, NH), lambda i: (0, 0)),
            pl.BlockSpec((MB, 1), lambda i: (i, 0)),
        ],
        out_specs=pl.BlockSpec((MB, NH), lambda i: (i, 0)),
        out_shape=jax.ShapeDtypeStruct((N, NH), jnp.float32),
    )(s, w2, dinv2col)


def _tc_combine(aggp, hs, dinv, b, wsc_row):
    def body(aggp_ref, hs_ref, d_ref, b_ref, wsc_ref, h_ref, ps_ref):
        a = aggp_ref[...]
        agg = a[0] + a[1]
        d = d_ref[...]
        base = agg[0:N] + hs_ref[...]
        hval = jnp.maximum(base * d[0:N][:, None] + b_ref[...][None, :], 0.0)
        hfull = jnp.concatenate(
            [hval, jnp.zeros((NROW - N, NH), jnp.float32)], axis=0)
        h_ref[...] = hfull
        p = jnp.sum(hfull * wsc_ref[...], axis=1)
        ps_ref[...] = d * p

    return pl.pallas_call(
        body,
        out_shape=(
            jax.ShapeDtypeStruct((NROW, NH), jnp.float32),
            jax.ShapeDtypeStruct((NROW,), jnp.float32),
        ),
    )(aggp, hs, dinv, b, wsc_row)


def _tc_readout(sagp, dinv, ps, bsc, h):
    def body(sagp_ref, d_ref, ps_ref, bsc_ref, h_ref, out_ref):
        sagg = jnp.sum(sagp_ref[...], axis=0)
        d = d_ref[...]
        score = d * (sagg + ps_ref[...]) + bsc_ref[0]

        bits = lax.bitcast_convert_type(score, jnp.uint32)
        key = jnp.where(bits >> 31 != 0, ~bits,
                        bits | jnp.uint32(0x80000000))
        idx = lax.broadcasted_iota(jnp.int32, (NROW,), 0)
        key = jnp.where(idx < N, key, jnp.uint32(0))

        t = jnp.uint32(0)
        for bit in range(31, -1, -1):
            cand = t | jnp.uint32(1 << bit)
            cnt = jnp.sum((key >= cand).astype(jnp.int32))
            t = jnp.where(cnt >= KTOP, cand, t)
        c_gt = jnp.sum((key > t).astype(jnp.int32))

        ties = (key == t)
        jcut = jnp.int32(0)
        for bit in range(13, -1, -1):
            cand = jcut + jnp.int32(1 << bit)
            f = c_gt + jnp.sum((ties & (idx < cand)).astype(jnp.int32))
            jcut = jnp.where(f < KTOP, cand, jcut)
        mask = (key > t) | (ties & (idx < jcut + 1))

        g = jnp.tanh(score)
        gated = h_ref[...] * g[:, None]
        mcol = mask.astype(jnp.float32)[:, None]
        xmax = jnp.max(jnp.where(mcol > 0.0, gated, -jnp.inf), axis=0)
        xsum = jnp.sum(gated * mcol, axis=0)
        out_ref[0, :] = xmax
        out_ref[1, :] = xsum / KTOP

    return pl.pallas_call(
        body,
        out_shape=jax.ShapeDtypeStruct((2, NH), jnp.float32),
    )(sagp, dinv, ps, bsc, h)


def _tc_head(x1p, x2p, lin1w, lin1b, lin3w, lin3b):
    def body(x1_ref, x2_ref, w1_ref, b1_ref, w3_ref, b3_ref, out_ref):
        z = jnp.concatenate(
            [x1_ref[0], x1_ref[1], x2_ref[0], x2_ref[1]], axis=0)
        zm = jnp.sum(z[:, None] * w1_ref[...], axis=0) + b1_ref[...]
        zm = jnp.maximum(zm, 0.0)
        o = jnp.sum(zm[:, None] * w3_ref[...], axis=0) + b3_ref[...]
        m = jnp.max(o)
        e = o - m
        out_ref[...] = (e - jnp.log(jnp.sum(jnp.exp(e))))[None, :]

    return pl.pallas_call(
        body,
        out_shape=jax.ShapeDtypeStruct((1, NCLS), jnp.float32),
    )(x1p, x2p, lin1w, lin1b, lin3w, lin3b)


def _pad_edges(ei):
    pad = EPAD - E
    pad_dst = N + jnp.arange(pad, dtype=jnp.int32) % (NROW - N)
    srcp = jnp.concatenate(
        [ei[0], jnp.zeros((pad,), jnp.int32)]).reshape(ECH, CHUNK)
    dstp = jnp.concatenate([ei[1], pad_dst]).reshape(ECH, CHUNK)
    return srcp, dstp


def kernel(x, s, edge1_index, edge2_index, batch, W1, b1, Wsc1, bsc1,
           W2, b2, Wsc2, bsc2, lin1_W, lin1_b, lin3_W, lin3_b):
    src1r, dst1r = _pad_edges(edge1_index)
    src2r, dst2r = _pad_edges(edge2_index)

    degp = _sc_degrees(dst1r, dst2r)
    dinv1, dinv2, hs1 = _tc_prep(degp, x, W1)
    hs2 = _tc_big_matmul(s, W2, dinv2.reshape(NROW, 1))

    aggp1 = _sc_row_agg(hs1, src1r, dst1r)
    h1, ps1 = _tc_combine(aggp1, hs1, dinv1, b1, Wsc1.reshape(1, NH))
    sagp1 = _sc_scalar_agg(ps1, src1r, dst1r)
    x1p = _tc_readout(sagp1, dinv1, ps1, bsc1, h1)

    aggp2 = _sc_row_agg(hs2, src2r, dst2r)
    h2, ps2 = _tc_combine(aggp2, hs2, dinv2, b2, Wsc2.reshape(1, NH))
    sagp2 = _sc_scalar_agg(ps2, src2r, dst2r)
    x2p = _tc_readout(sagp2, dinv2, ps2, bsc2, h2)

    return _tc_head(x1p, x2p, lin1_W, lin1_b, lin3_W, lin3_b)


# speedup vs baseline: 15.6100x; 1.0063x over previous
import functools

import jax
import jax.numpy as jnp
from jax import lax
from jax.experimental import pallas as pl
from jax.experimental.pallas import tpu as pltpu
from jax.experimental.pallas import tpu_sc as plsc

N = 10000
D = 128
NH = 128
E = 320000
NCLS = 10
KTOP = 5000

NCORES = 2
NSUB = 16
NW = NCORES * NSUB
CHUNK = 128
CPW = 80
EPAD = NW * CPW * CHUNK
ECH = EPAD // CHUNK
NROW = 10240
DUMMY = 10016
RPS = NROW // NSUB
NBUF = 2
CPW0 = 120
CPW1 = 40

_HI = lax.Precision.HIGHEST


def _mesh():
    return plsc.VectorSubcoreMesh(core_axis_name="c", subcore_axis_name="s")


_SC_PARAMS = pltpu.CompilerParams(use_tc_tiling_on_sc=False,
                                  needs_layout_passes=False)


def _sc_degrees(dst1r, dst2r):
    @functools.partial(
        pl.kernel,
        out_type=jax.ShapeDtypeStruct((NW, 2, NROW), jnp.float32),
        mesh=_mesh(),
        compiler_params=_SC_PARAMS,
        scratch_types=[
            pltpu.VMEM((CPW, CHUNK), jnp.int32),
            pltpu.VMEM((CPW, CHUNK), jnp.int32),
            pltpu.VMEM((NROW,), jnp.float32),
            pltpu.VMEM((NROW,), jnp.float32),
        ],
    )
    def deg_kernel(d1_hbm, d2_hbm, out_hbm, d1_v, d2_v, a1_v, a2_v):
        cid = lax.axis_index("c")
        sid = lax.axis_index("s")
        wid = sid * NCORES + cid
        base = wid * CPW
        pltpu.sync_copy(d1_hbm.at[pl.ds(base, CPW)], d1_v)
        pltpu.sync_copy(d2_hbm.at[pl.ds(base, CPW)], d2_v)
        z16 = jnp.zeros((16,), jnp.float32)
        ones = jnp.ones((16,), jnp.float32)

        def zbody(i, carry):
            a1_v[pl.ds(i * 16, 16)] = z16
            a2_v[pl.ds(i * 16, 16)] = z16
            return carry

        lax.fori_loop(0, NROW // 16, zbody, 0)

        def ebody(i, carry):
            r = i // (CHUNK // 16)
            c = (i % (CHUNK // 16)) * 16
            plsc.addupdate_scatter(a1_v, [d1_v[r, pl.ds(c, 16)]], ones)
            plsc.addupdate_scatter(a2_v, [d2_v[r, pl.ds(c, 16)]], ones)
            return carry

        lax.fori_loop(0, CPW * (CHUNK // 16), ebody, 0)
        pltpu.sync_copy(a1_v, out_hbm.at[wid, 0])
        pltpu.sync_copy(a2_v, out_hbm.at[wid, 1])

    return deg_kernel(dst1r, dst2r)


def _sc_row_agg(hs, srcr, dstr):
    @functools.partial(
        pl.kernel,
        out_type=jax.ShapeDtypeStruct((NCORES, NROW, NH), jnp.float32),
        mesh=_mesh(),
        compiler_params=_SC_PARAMS,
        scratch_types=[
            pltpu.VMEM((CPW0, CHUNK), jnp.int32),
            pltpu.VMEM((NBUF, CHUNK), jnp.int32),
            pltpu.VMEM((NBUF, CHUNK, NH), jnp.float32),
            pltpu.VMEM_SHARED((NROW, NH), jnp.float32),
            [pltpu.SemaphoreType.DMA] * NBUF,
            [pltpu.SemaphoreType.DMA] * NBUF,
        ],
    )
    def rowagg_kernel(hs_hbm, src_hbm, dst_hbm, out_hbm,
                      src_v, dstr_v, rows_v, acc_sh, gsems, dsems):
        cid = lax.axis_index("c")
        sid = lax.axis_index("s")
        base = sid * (CPW0 + CPW1) + cid * CPW0
        cpw_my = jnp.where(cid == 0, CPW0, CPW1)

        @pl.when(cid == 0)
        def _():
            pltpu.sync_copy(src_hbm.at[pl.ds(base, CPW0)],
                            src_v.at[pl.ds(0, CPW0)])

        @pl.when(cid == 1)
        def _():
            pltpu.sync_copy(src_hbm.at[pl.ds(base, CPW1)],
                            src_v.at[pl.ds(0, CPW1)])

        z16 = jnp.zeros((16,), jnp.float32)

        def zb_body(i, carry):
            rows_v[0, i // 8, pl.ds((i % 8) * 16, 16)] = z16
            return carry

        lax.fori_loop(0, CHUNK * (NH // 16), zb_body, 0)
        row0 = sid * RPS

        def zacc_body(t, carry):
            pltpu.sync_copy(rows_v.at[0],
                            acc_sh.at[pl.ds(row0 + t * CHUNK, CHUNK)])
            return carry

        lax.fori_loop(0, RPS // CHUNK, zacc_body, 0)
        plsc.subcore_barrier()

        def start_chunk(j, b):
            pltpu.async_copy(dst_hbm.at[base + j], dstr_v.at[b], dsems[b])
            pltpu.async_copy(hs_hbm.at[src_v.at[j]], rows_v.at[b], gsems[b])

        def wait_chunk(b):
            pltpu.make_async_copy(dst_hbm.at[0], dstr_v.at[b],
                                  dsems[b]).wait()
            pltpu.make_async_copy(hs_hbm.at[pl.ds(0, CHUNK)],
                                  rows_v.at[b], gsems[b]).wait()

        for b in range(NBUF):
            start_chunk(b, b)

        def ebody(g, carry):
            for b in range(NBUF):
                j = g * NBUF + b
                wait_chunk(b)
                pltpu.sync_copy(rows_v.at[b], acc_sh.at[dstr_v.at[b]],
                                add=True)
                jn = j + NBUF

                @pl.when(jn < cpw_my)
                def _():
                    start_chunk(jn, b)

            return carry

        lax.fori_loop(0, cpw_my // NBUF, ebody, 0)
        plsc.subcore_barrier()
        pltpu.sync_copy(acc_sh.at[pl.ds(row0, RPS)],
                        out_hbm.at[cid, pl.ds(row0, RPS)])

    return rowagg_kernel(hs, srcr, dstr)


def _sc_scalar_agg(tab, srcr, dstr):
    @functools.partial(
        pl.kernel,
        out_type=jax.ShapeDtypeStruct((NW, NROW), jnp.float32),
        mesh=_mesh(),
        compiler_params=_SC_PARAMS,
        scratch_types=[
            pltpu.VMEM((CPW, CHUNK), jnp.int32),
            pltpu.VMEM((CPW, CHUNK), jnp.int32),
            pltpu.VMEM((NROW,), jnp.float32),
            pltpu.VMEM((NROW,), jnp.float32),
        ],
    )
    def scal_kernel(tab_hbm, src_hbm, dst_hbm, out_hbm,
                    src_v, dst_v, tab_v, acc_v):
        cid = lax.axis_index("c")
        sid = lax.axis_index("s")
        wid = sid * NCORES + cid
        base = wid * CPW
        pltpu.sync_copy(src_hbm.at[pl.ds(base, CPW)], src_v)
        pltpu.sync_copy(dst_hbm.at[pl.ds(base, CPW)], dst_v)
        pltpu.sync_copy(tab_hbm, tab_v)
        z16 = jnp.zeros((16,), jnp.float32)

        def zbody(i, carry):
            acc_v[pl.ds(i * 16, 16)] = z16
            return carry

        lax.fori_loop(0, NROW // 16, zbody, 0)

        def ebody(i, carry):
            r = i // (CHUNK // 16)
            c = (i % (CHUNK // 16)) * 16
            vals = plsc.load_gather(tab_v, [src_v[r, pl.ds(c, 16)]])
            plsc.addupdate_scatter(acc_v, [dst_v[r, pl.ds(c, 16)]], vals)
            return carry

        lax.fori_loop(0, CPW * (CHUNK // 16), ebody, 0)
        pltpu.sync_copy(acc_v, out_hbm.at[wid])

    return scal_kernel(tab, srcr, dstr)


def _tc_prep(degp, x, w1):
    def body(degp_ref, x_ref, w1_ref, d1_ref, d2_ref, hs1_ref):
        deg = jnp.sum(degp_ref[...], axis=0) + 1.0
        dinv = lax.rsqrt(deg)
        d1 = dinv[0]
        d2 = dinv[1]
        d1_ref[...] = d1
        d2_ref[...] = d2
        h0 = jnp.dot(x_ref[...], w1_ref[...],
                     preferred_element_type=jnp.float32, precision=_HI)
        hs1_ref[...] = h0 * d1[0:N][:, None]

    return pl.pallas_call(
        body,
        out_shape=(
            jax.ShapeDtypeStruct((NROW,), jnp.float32),
            jax.ShapeDtypeStruct((NROW,), jnp.float32),
            jax.ShapeDtypeStruct((N, NH), jnp.float32),
        ),
    )(degp, x, w1)


def _tc_big_matmul(s, w2, dinv2col):
    MB = 200
    nm = N // MB

    def body(s_ref, w_ref, d_ref, o_ref):
        o_ref[...] = jnp.dot(s_ref[...], w_ref[...],
                             preferred_element_type=jnp.float32,
                             precision=_HI) * d_ref[...]

    return pl.pallas_call(
        body,
        grid=(nm,),
        in_specs=[
            pl.BlockSpec((MB, N), lambda i: (i, 0)),
            pl.BlockSpec((N, NH), lambda i: (0, 0)),
            pl.BlockSpec((MB, 1), lambda i: (i, 0)),
        ],
        out_specs=pl.BlockSpec((MB, NH), lambda i: (i, 0)),
        out_shape=jax.ShapeDtypeStruct((N, NH), jnp.float32),
    )(s, w2, dinv2col)


def _tc_combine(aggp, hs, dinv, b, wsc_row):
    def body(aggp_ref, hs_ref, d_ref, b_ref, wsc_ref, h_ref, ps_ref):
        a = aggp_ref[...]
        agg = a[0] + a[1]
        d = d_ref[...]
        base = agg[0:N] + hs_ref[...]
        hval = jnp.maximum(base * d[0:N][:, None] + b_ref[...][None, :], 0.0)
        hfull = jnp.concatenate(
            [hval, jnp.zeros((NROW - N, NH), jnp.float32)], axis=0)
        h_ref[...] = hfull
        p = jnp.sum(hfull * wsc_ref[...], axis=1)
        ps_ref[...] = d * p

    return pl.pallas_call(
        body,
        out_shape=(
            jax.ShapeDtypeStruct((NROW, NH), jnp.float32),
            jax.ShapeDtypeStruct((NROW,), jnp.float32),
        ),
    )(aggp, hs, dinv, b, wsc_row)


def _tc_readout(sagp, dinv, ps, bsc, h):
    def body(sagp_ref, d_ref, ps_ref, bsc_ref, h_ref, out_ref):
        sagg = jnp.sum(sagp_ref[...], axis=0)
        d = d_ref[...]
        score = d * (sagg + ps_ref[...]) + bsc_ref[0]

        bits = lax.bitcast_convert_type(score, jnp.uint32)
        key = jnp.where(bits >> 31 != 0, ~bits,
                        bits | jnp.uint32(0x80000000))
        idx = lax.broadcasted_iota(jnp.int32, (NROW,), 0)
        key = jnp.where(idx < N, key, jnp.uint32(0))

        t = jnp.uint32(0)
        for bit in range(31, -1, -1):
            cand = t | jnp.uint32(1 << bit)
            cnt = jnp.sum((key >= cand).astype(jnp.int32))
            t = jnp.where(cnt >= KTOP, cand, t)
        c_gt = jnp.sum((key > t).astype(jnp.int32))

        ties = (key == t)
        jcut = jnp.int32(0)
        for bit in range(13, -1, -1):
            cand = jcut + jnp.int32(1 << bit)
            f = c_gt + jnp.sum((ties & (idx < cand)).astype(jnp.int32))
            jcut = jnp.where(f < KTOP, cand, jcut)
        mask = (key > t) | (ties & (idx < jcut + 1))

        g = jnp.tanh(score)
        gated = h_ref[...] * g[:, None]
        mcol = mask.astype(jnp.float32)[:, None]
        xmax = jnp.max(jnp.where(mcol > 0.0, gated, -jnp.inf), axis=0)
        xsum = jnp.sum(gated * mcol, axis=0)
        out_ref[0, :] = xmax
        out_ref[1, :] = xsum / KTOP

    return pl.pallas_call(
        body,
        out_shape=jax.ShapeDtypeStruct((2, NH), jnp.float32),
    )(sagp, dinv, ps, bsc, h)


def _tc_head(x1p, x2p, lin1w, lin1b, lin3w, lin3b):
    def body(x1_ref, x2_ref, w1_ref, b1_ref, w3_ref, b3_ref, out_ref):
        z = jnp.concatenate(
            [x1_ref[0], x1_ref[1], x2_ref[0], x2_ref[1]], axis=0)
        zm = jnp.sum(z[:, None] * w1_ref[...], axis=0) + b1_ref[...]
        zm = jnp.maximum(zm, 0.0)
        o = jnp.sum(zm[:, None] * w3_ref[...], axis=0) + b3_ref[...]
        m = jnp.max(o)
        e = o - m
        out_ref[...] = (e - jnp.log(jnp.sum(jnp.exp(e))))[None, :]

    return pl.pallas_call(
        body,
        out_shape=jax.ShapeDtypeStruct((1, NCLS), jnp.float32),
    )(x1p, x2p, lin1w, lin1b, lin3w, lin3b)


def _pad_edges(ei):
    pad = EPAD - E
    pad_dst = N + jnp.arange(pad, dtype=jnp.int32) % (NROW - N)
    srcp = jnp.concatenate(
        [ei[0], jnp.zeros((pad,), jnp.int32)]).reshape(ECH, CHUNK)
    dstp = jnp.concatenate([ei[1], pad_dst]).reshape(ECH, CHUNK)
    return srcp, dstp


def kernel(x, s, edge1_index, edge2_index, batch, W1, b1, Wsc1, bsc1,
           W2, b2, Wsc2, bsc2, lin1_W, lin1_b, lin3_W, lin3_b):
    src1r, dst1r = _pad_edges(edge1_index)
    src2r, dst2r = _pad_edges(edge2_index)

    degp = _sc_degrees(dst1r, dst2r)
    dinv1, dinv2, hs1 = _tc_prep(degp, x, W1)
    hs2 = _tc_big_matmul(s, W2, dinv2.reshape(NROW, 1))

    aggp1 = _sc_row_agg(hs1, src1r, dst1r)
    h1, ps1 = _tc_combine(aggp1, hs1, dinv1, b1, Wsc1.reshape(1, NH))
    sagp1 = _sc_scalar_agg(ps1, src1r, dst1r)
    x1p = _tc_readout(sagp1, dinv1, ps1, bsc1, h1)

    aggp2 = _sc_row_agg(hs2, src2r, dst2r)
    h2, ps2 = _tc_combine(aggp2, hs2, dinv2, b2, Wsc2.reshape(1, NH))
    sagp2 = _sc_scalar_agg(ps2, src2r, dst2r)
    x2p = _tc_readout(sagp2, dinv2, ps2, bsc2, h2)

    return _tc_head(x1p, x2p, lin1_W, lin1_b, lin3_W, lin3_b)


# speedup vs baseline: 15.8004x; 1.0122x over previous
import functools

import jax
import jax.numpy as jnp
from jax import lax
from jax.experimental import pallas as pl
from jax.experimental.pallas import tpu as pltpu
from jax.experimental.pallas import tpu_sc as plsc

N = 10000
D = 128
NH = 128
E = 320000
NCLS = 10
KTOP = 5000

NCORES = 2
NSUB = 16
NW = NCORES * NSUB
CHUNK = 128
CPW = 80
EPAD = NW * CPW * CHUNK
ECH = EPAD // CHUNK
NROW = 10240
DUMMY = 10016
RPS = NROW // NSUB
NBUF = 2
CPW0 = 80
CPW1 = 80

_HI = lax.Precision.HIGHEST


def _mesh():
    return plsc.VectorSubcoreMesh(core_axis_name="c", subcore_axis_name="s")


_SC_PARAMS = pltpu.CompilerParams(use_tc_tiling_on_sc=False,
                                  needs_layout_passes=False)


def _sc_degrees(dst1r, dst2r):
    @functools.partial(
        pl.kernel,
        out_type=jax.ShapeDtypeStruct((NW, 2, NROW), jnp.float32),
        mesh=_mesh(),
        compiler_params=_SC_PARAMS,
        scratch_types=[
            pltpu.VMEM((CPW, CHUNK), jnp.int32),
            pltpu.VMEM((CPW, CHUNK), jnp.int32),
            pltpu.VMEM((NROW,), jnp.float32),
            pltpu.VMEM((NROW,), jnp.float32),
        ],
    )
    def deg_kernel(d1_hbm, d2_hbm, out_hbm, d1_v, d2_v, a1_v, a2_v):
        cid = lax.axis_index("c")
        sid = lax.axis_index("s")
        wid = sid * NCORES + cid
        base = wid * CPW
        pltpu.sync_copy(d1_hbm.at[pl.ds(base, CPW)], d1_v)
        pltpu.sync_copy(d2_hbm.at[pl.ds(base, CPW)], d2_v)
        z16 = jnp.zeros((16,), jnp.float32)
        ones = jnp.ones((16,), jnp.float32)

        def zbody(i, carry):
            a1_v[pl.ds(i * 16, 16)] = z16
            a2_v[pl.ds(i * 16, 16)] = z16
            return carry

        lax.fori_loop(0, NROW // 16, zbody, 0)

        def ebody(i, carry):
            r = i // (CHUNK // 16)
            c = (i % (CHUNK // 16)) * 16
            plsc.addupdate_scatter(a1_v, [d1_v[r, pl.ds(c, 16)]], ones)
            plsc.addupdate_scatter(a2_v, [d2_v[r, pl.ds(c, 16)]], ones)
            return carry

        lax.fori_loop(0, CPW * (CHUNK // 16), ebody, 0)
        pltpu.sync_copy(a1_v, out_hbm.at[wid, 0])
        pltpu.sync_copy(a2_v, out_hbm.at[wid, 1])

    return deg_kernel(dst1r, dst2r)


def _sc_row_agg(hs, srcr, dstr):
    @functools.partial(
        pl.kernel,
        out_type=jax.ShapeDtypeStruct((NCORES, NROW, NH), jnp.float32),
        mesh=_mesh(),
        compiler_params=_SC_PARAMS,
        scratch_types=[
            pltpu.VMEM((CPW0, CHUNK), jnp.int32),
            pltpu.VMEM((NBUF, CHUNK), jnp.int32),
            pltpu.VMEM((NBUF, CHUNK, NH), jnp.float32),
            pltpu.VMEM_SHARED((NROW, NH), jnp.float32),
            [pltpu.SemaphoreType.DMA] * NBUF,
            [pltpu.SemaphoreType.DMA] * NBUF,
        ],
    )
    def rowagg_kernel(hs_hbm, src_hbm, dst_hbm, out_hbm,
                      src_v, dstr_v, rows_v, acc_sh, gsems, dsems):
        cid = lax.axis_index("c")
        sid = lax.axis_index("s")
        base = sid * (CPW0 + CPW1) + cid * CPW0
        cpw_my = jnp.where(cid == 0, CPW0, CPW1)

        @pl.when(cid == 0)
        def _():
            pltpu.sync_copy(src_hbm.at[pl.ds(base, CPW0)],
                            src_v.at[pl.ds(0, CPW0)])

        @pl.when(cid == 1)
        def _():
            pltpu.sync_copy(src_hbm.at[pl.ds(base, CPW1)],
                            src_v.at[pl.ds(0, CPW1)])

        z16 = jnp.zeros((16,), jnp.float32)

        def zb_body(i, carry):
            rows_v[0, i // 8, pl.ds((i % 8) * 16, 16)] = z16
            return carry

        lax.fori_loop(0, CHUNK * (NH // 16), zb_body, 0)
        row0 = sid * RPS

        def zacc_body(t, carry):
            pltpu.sync_copy(rows_v.at[0],
                            acc_sh.at[pl.ds(row0 + t * CHUNK, CHUNK)])
            return carry

        lax.fori_loop(0, RPS // CHUNK, zacc_body, 0)
        plsc.subcore_barrier()

        def start_chunk(j, b):
            pltpu.async_copy(dst_hbm.at[base + j], dstr_v.at[b], dsems[b])
            pltpu.async_copy(hs_hbm.at[src_v.at[j]], rows_v.at[b], gsems[b])

        def wait_chunk(b):
            pltpu.make_async_copy(dst_hbm.at[0], dstr_v.at[b],
                                  dsems[b]).wait()
            pltpu.make_async_copy(hs_hbm.at[pl.ds(0, CHUNK)],
                                  rows_v.at[b], gsems[b]).wait()

        for b in range(NBUF):
            start_chunk(b, b)

        def ebody(g, carry):
            for b in range(NBUF):
                j = g * NBUF + b
                wait_chunk(b)
                pltpu.sync_copy(rows_v.at[b], acc_sh.at[dstr_v.at[b]],
                                add=True)
                jn = j + NBUF

                @pl.when(jn < cpw_my)
                def _():
                    start_chunk(jn, b)

            return carry

        lax.fori_loop(0, cpw_my // NBUF, ebody, 0)
        plsc.subcore_barrier()
        pltpu.sync_copy(acc_sh.at[pl.ds(row0, RPS)],
                        out_hbm.at[cid, pl.ds(row0, RPS)])

    return rowagg_kernel(hs, srcr, dstr)


def _sc_scalar_agg(tab, srcr, dstr):
    @functools.partial(
        pl.kernel,
        out_type=jax.ShapeDtypeStruct((NW, NROW), jnp.float32),
        mesh=_mesh(),
        compiler_params=_SC_PARAMS,
        scratch_types=[
            pltpu.VMEM((CPW, CHUNK), jnp.int32),
            pltpu.VMEM((CPW, CHUNK), jnp.int32),
            pltpu.VMEM((NROW,), jnp.float32),
            pltpu.VMEM((NROW,), jnp.float32),
        ],
    )
    def scal_kernel(tab_hbm, src_hbm, dst_hbm, out_hbm,
                    src_v, dst_v, tab_v, acc_v):
        cid = lax.axis_index("c")
        sid = lax.axis_index("s")
        wid = sid * NCORES + cid
        base = wid * CPW
        pltpu.sync_copy(src_hbm.at[pl.ds(base, CPW)], src_v)
        pltpu.sync_copy(dst_hbm.at[pl.ds(base, CPW)], dst_v)
        pltpu.sync_copy(tab_hbm, tab_v)
        z16 = jnp.zeros((16,), jnp.float32)

        def zbody(i, carry):
            acc_v[pl.ds(i * 16, 16)] = z16
            return carry

        lax.fori_loop(0, NROW // 16, zbody, 0)

        def ebody(i, carry):
            r = i // (CHUNK // 16)
            c = (i % (CHUNK // 16)) * 16
            vals = plsc.load_gather(tab_v, [src_v[r, pl.ds(c, 16)]])
            plsc.addupdate_scatter(acc_v, [dst_v[r, pl.ds(c, 16)]], vals)
            return carry

        lax.fori_loop(0, CPW * (CHUNK // 16), ebody, 0)
        pltpu.sync_copy(acc_v, out_hbm.at[wid])

    return scal_kernel(tab, srcr, dstr)


def _tc_prep(degp, x, w1):
    def body(degp_ref, x_ref, w1_ref, d1_ref, d2_ref, hs1_ref):
        deg = jnp.sum(degp_ref[...], axis=0) + 1.0
        dinv = lax.rsqrt(deg)
        d1 = dinv[0]
        d2 = dinv[1]
        d1_ref[...] = d1
        d2_ref[...] = d2
        h0 = jnp.dot(x_ref[...], w1_ref[...],
                     preferred_element_type=jnp.float32, precision=_HI)
        hs1_ref[...] = h0 * d1[0:N][:, None]

    return pl.pallas_call(
        body,
        out_shape=(
            jax.ShapeDtypeStruct((NROW,), jnp.float32),
            jax.ShapeDtypeStruct((NROW,), jnp.float32),
            jax.ShapeDtypeStruct((N, NH), jnp.float32),
        ),
    )(degp, x, w1)


def _tc_big_matmul(s, w2, dinv2col):
    MB = 200
    nm = N // MB

    def body(s_ref, w_ref, d_ref, o_ref):
        o_ref[...] = jnp.dot(s_ref[...], w_ref[...],
                             preferred_element_type=jnp.float32) * d_ref[...]

    return pl.pallas_call(
        body,
        grid=(nm,),
        in_specs=[
            pl.BlockSpec((MB, N), lambda i: (i, 0)),
            pl.BlockSpec((N, NH), lambda i: (0, 0)),
            pl.BlockSpec((MB, 1), lambda i: (i, 0)),
        ],
        out_specs=pl.BlockSpec((MB, NH), lambda i: (i, 0)),
        out_shape=jax.ShapeDtypeStruct((N, NH), jnp.float32),
    )(s, w2, dinv2col)


def _tc_combine(aggp, hs, dinv, b, wsc_row):
    def body(aggp_ref, hs_ref, d_ref, b_ref, wsc_ref, h_ref, ps_ref):
        a = aggp_ref[...]
        agg = a[0] + a[1]
        d = d_ref[...]
        base = agg[0:N] + hs_ref[...]
        hval = jnp.maximum(base * d[0:N][:, None] + b_ref[...][None, :], 0.0)
        hfull = jnp.concatenate(
            [hval, jnp.zeros((NROW - N, NH), jnp.float32)], axis=0)
        h_ref[...] = hfull
        p = jnp.sum(hfull * wsc_ref[...], axis=1)
        ps_ref[...] = d * p

    return pl.pallas_call(
        body,
        out_shape=(
            jax.ShapeDtypeStruct((NROW, NH), jnp.float32),
            jax.ShapeDtypeStruct((NROW,), jnp.float32),
        ),
    )(aggp, hs, dinv, b, wsc_row)


def _tc_readout(sagp, dinv, ps, bsc, h):
    def body(sagp_ref, d_ref, ps_ref, bsc_ref, h_ref, out_ref):
        sagg = jnp.sum(sagp_ref[...], axis=0)
        d = d_ref[...]
        score = d * (sagg + ps_ref[...]) + bsc_ref[0]

        bits = lax.bitcast_convert_type(score, jnp.uint32)
        key = jnp.where(bits >> 31 != 0, ~bits,
                        bits | jnp.uint32(0x80000000))
        idx = lax.broadcasted_iota(jnp.int32, (NROW,), 0)
        key = jnp.where(idx < N, key, jnp.uint32(0))

        t = jnp.uint32(0)
        for bit in range(31, -1, -1):
            cand = t | jnp.uint32(1 << bit)
            cnt = jnp.sum((key >= cand).astype(jnp.int32))
            t = jnp.where(cnt >= KTOP, cand, t)
        c_gt = jnp.sum((key > t).astype(jnp.int32))

        ties = (key == t)
        jcut = jnp.int32(0)
        for bit in range(13, -1, -1):
            cand = jcut + jnp.int32(1 << bit)
            f = c_gt + jnp.sum((ties & (idx < cand)).astype(jnp.int32))
            jcut = jnp.where(f < KTOP, cand, jcut)
        mask = (key > t) | (ties & (idx < jcut + 1))

        g = jnp.tanh(score)
        gated = h_ref[...] * g[:, None]
        mcol = mask.astype(jnp.float32)[:, None]
        xmax = jnp.max(jnp.where(mcol > 0.0, gated, -jnp.inf), axis=0)
        xsum = jnp.sum(gated * mcol, axis=0)
        out_ref[0, :] = xmax
        out_ref[1, :] = xsum / KTOP

    return pl.pallas_call(
        body,
        out_shape=jax.ShapeDtypeStruct((2, NH), jnp.float32),
    )(sagp, dinv, ps, bsc, h)


def _tc_head(x1p, x2p, lin1w, lin1b, lin3w, lin3b):
    def body(x1_ref, x2_ref, w1_ref, b1_ref, w3_ref, b3_ref, out_ref):
        z = jnp.concatenate(
            [x1_ref[0], x1_ref[1], x2_ref[0], x2_ref[1]], axis=0)
        zm = jnp.sum(z[:, None] * w1_ref[...], axis=0) + b1_ref[...]
        zm = jnp.maximum(zm, 0.0)
        o = jnp.sum(zm[:, None] * w3_ref[...], axis=0) + b3_ref[...]
        m = jnp.max(o)
        e = o - m
        out_ref[...] = (e - jnp.log(jnp.sum(jnp.exp(e))))[None, :]

    return pl.pallas_call(
        body,
        out_shape=jax.ShapeDtypeStruct((1, NCLS), jnp.float32),
    )(x1p, x2p, lin1w, lin1b, lin3w, lin3b)


def _pad_edges(ei):
    pad = EPAD - E
    pad_dst = N + jnp.arange(pad, dtype=jnp.int32) % (NROW - N)
    srcp = jnp.concatenate(
        [ei[0], jnp.zeros((pad,), jnp.int32)]).reshape(ECH, CHUNK)
    dstp = jnp.concatenate([ei[1], pad_dst]).reshape(ECH, CHUNK)
    return srcp, dstp


def kernel(x, s, edge1_index, edge2_index, batch, W1, b1, Wsc1, bsc1,
           W2, b2, Wsc2, bsc2, lin1_W, lin1_b, lin3_W, lin3_b):
    src1r, dst1r = _pad_edges(edge1_index)
    src2r, dst2r = _pad_edges(edge2_index)

    degp = _sc_degrees(dst1r, dst2r)
    dinv1, dinv2, hs1 = _tc_prep(degp, x, W1)
    hs2 = _tc_big_matmul(s, W2, dinv2.reshape(NROW, 1))

    aggp1 = _sc_row_agg(hs1, src1r, dst1r)
    h1, ps1 = _tc_combine(aggp1, hs1, dinv1, b1, Wsc1.reshape(1, NH))
    sagp1 = _sc_scalar_agg(ps1, src1r, dst1r)
    x1p = _tc_readout(sagp1, dinv1, ps1, bsc1, h1)

    aggp2 = _sc_row_agg(hs2, src2r, dst2r)
    h2, ps2 = _tc_combine(aggp2, hs2, dinv2, b2, Wsc2.reshape(1, NH))
    sagp2 = _sc_scalar_agg(ps2, src2r, dst2r)
    x2p = _tc_readout(sagp2, dinv2, ps2, bsc2, h2)

    return _tc_head(x1p, x2p, lin1_W, lin1_b, lin3_W, lin3_b)


# speedup vs baseline: 16.0107x; 1.0133x over previous
import functools

import jax
import jax.numpy as jnp
from jax import lax
from jax.experimental import pallas as pl
from jax.experimental.pallas import tpu as pltpu
from jax.experimental.pallas import tpu_sc as plsc

N = 10000
D = 128
NH = 128
E = 320000
NCLS = 10
KTOP = 5000

NCORES = 2
NSUB = 16
NW = NCORES * NSUB
CHUNK = 128
CPW = 80
EPAD = NW * CPW * CHUNK
ECH = EPAD // CHUNK
NROW = 10240
DUMMY = 10016
RPS = NROW // NSUB
NBUF = 4
RCHUNK = 64
RCPW = EPAD // RCHUNK // NW

_HI = lax.Precision.HIGHEST


def _mesh():
    return plsc.VectorSubcoreMesh(core_axis_name="c", subcore_axis_name="s")


_SC_PARAMS = pltpu.CompilerParams(use_tc_tiling_on_sc=False,
                                  needs_layout_passes=False)


def _sc_degrees(dst1r, dst2r):
    @functools.partial(
        pl.kernel,
        out_type=jax.ShapeDtypeStruct((NW, 2, NROW), jnp.float32),
        mesh=_mesh(),
        compiler_params=_SC_PARAMS,
        scratch_types=[
            pltpu.VMEM((CPW, CHUNK), jnp.int32),
            pltpu.VMEM((CPW, CHUNK), jnp.int32),
            pltpu.VMEM((NROW,), jnp.float32),
            pltpu.VMEM((NROW,), jnp.float32),
        ],
    )
    def deg_kernel(d1_hbm, d2_hbm, out_hbm, d1_v, d2_v, a1_v, a2_v):
        cid = lax.axis_index("c")
        sid = lax.axis_index("s")
        wid = sid * NCORES + cid
        base = wid * CPW
        pltpu.sync_copy(d1_hbm.at[pl.ds(base, CPW)], d1_v)
        pltpu.sync_copy(d2_hbm.at[pl.ds(base, CPW)], d2_v)
        z16 = jnp.zeros((16,), jnp.float32)
        ones = jnp.ones((16,), jnp.float32)

        def zbody(i, carry):
            a1_v[pl.ds(i * 16, 16)] = z16
            a2_v[pl.ds(i * 16, 16)] = z16
            return carry

        lax.fori_loop(0, NROW // 16, zbody, 0)

        def ebody(i, carry):
            r = i // (CHUNK // 16)
            c = (i % (CHUNK // 16)) * 16
            plsc.addupdate_scatter(a1_v, [d1_v[r, pl.ds(c, 16)]], ones)
            plsc.addupdate_scatter(a2_v, [d2_v[r, pl.ds(c, 16)]], ones)
            return carry

        lax.fori_loop(0, CPW * (CHUNK // 16), ebody, 0)
        pltpu.sync_copy(a1_v, out_hbm.at[wid, 0])
        pltpu.sync_copy(a2_v, out_hbm.at[wid, 1])

    return deg_kernel(dst1r, dst2r)


def _sc_row_agg(hs, srcr, dstr):
    @functools.partial(
        pl.kernel,
        out_type=jax.ShapeDtypeStruct((NCORES, NROW, NH), jnp.float32),
        mesh=_mesh(),
        compiler_params=_SC_PARAMS,
        scratch_types=[
            pltpu.VMEM((RCPW, RCHUNK), jnp.int32),
            pltpu.VMEM((NBUF, RCHUNK), jnp.int32),
            pltpu.VMEM((NBUF, RCHUNK, NH), jnp.float32),
            pltpu.VMEM_SHARED((NROW, NH), jnp.float32),
            [pltpu.SemaphoreType.DMA] * NBUF,
            [pltpu.SemaphoreType.DMA] * NBUF,
        ],
    )
    def rowagg_kernel(hs_hbm, src_hbm, dst_hbm, out_hbm,
                      src_v, dstr_v, rows_v, acc_sh, gsems, dsems):
        cid = lax.axis_index("c")
        sid = lax.axis_index("s")
        wid = sid * NCORES + cid
        base = wid * RCPW
        pltpu.sync_copy(src_hbm.at[pl.ds(base, RCPW)], src_v)
        z16 = jnp.zeros((16,), jnp.float32)

        def zb_body(i, carry):
            rows_v[0, i // 8, pl.ds((i % 8) * 16, 16)] = z16
            return carry

        lax.fori_loop(0, RCHUNK * (NH // 16), zb_body, 0)
        row0 = sid * RPS

        def zacc_body(t, carry):
            pltpu.sync_copy(rows_v.at[0],
                            acc_sh.at[pl.ds(row0 + t * RCHUNK, RCHUNK)])
            return carry

        lax.fori_loop(0, RPS // RCHUNK, zacc_body, 0)
        plsc.subcore_barrier()

        def start_chunk(j, b):
            pltpu.async_copy(dst_hbm.at[base + j], dstr_v.at[b], dsems[b])
            pltpu.async_copy(hs_hbm.at[src_v.at[j]], rows_v.at[b], gsems[b])

        def wait_chunk(b):
            pltpu.make_async_copy(dst_hbm.at[0], dstr_v.at[b],
                                  dsems[b]).wait()
            pltpu.make_async_copy(hs_hbm.at[pl.ds(0, RCHUNK)],
                                  rows_v.at[b], gsems[b]).wait()

        for b in range(NBUF):
            start_chunk(b, b)

        def ebody(g, carry):
            for b in range(NBUF):
                j = g * NBUF + b
                wait_chunk(b)
                pltpu.sync_copy(rows_v.at[b], acc_sh.at[dstr_v.at[b]],
                                add=True)
                jn = j + NBUF

                @pl.when(jn < RCPW)
                def _():
                    start_chunk(jn, b)

            return carry

        lax.fori_loop(0, RCPW // NBUF, ebody, 0)
        plsc.subcore_barrier()
        pltpu.sync_copy(acc_sh.at[pl.ds(row0, RPS)],
                        out_hbm.at[cid, pl.ds(row0, RPS)])

    return rowagg_kernel(hs, srcr, dstr)


def _sc_scalar_agg(tab, srcr, dstr):
    @functools.partial(
        pl.kernel,
        out_type=jax.ShapeDtypeStruct((NW, NROW), jnp.float32),
        mesh=_mesh(),
        compiler_params=_SC_PARAMS,
        scratch_types=[
            pltpu.VMEM((CPW, CHUNK), jnp.int32),
            pltpu.VMEM((CPW, CHUNK), jnp.int32),
            pltpu.VMEM((NROW,), jnp.float32),
            pltpu.VMEM((NROW,), jnp.float32),
        ],
    )
    def scal_kernel(tab_hbm, src_hbm, dst_hbm, out_hbm,
                    src_v, dst_v, tab_v, acc_v):
        cid = lax.axis_index("c")
        sid = lax.axis_index("s")
        wid = sid * NCORES + cid
        base = wid * CPW
        pltpu.sync_copy(src_hbm.at[pl.ds(base, CPW)], src_v)
        pltpu.sync_copy(dst_hbm.at[pl.ds(base, CPW)], dst_v)
        pltpu.sync_copy(tab_hbm, tab_v)
        z16 = jnp.zeros((16,), jnp.float32)

        def zbody(i, carry):
            acc_v[pl.ds(i * 16, 16)] = z16
            return carry

        lax.fori_loop(0, NROW // 16, zbody, 0)

        def ebody(i, carry):
            r = i // (CHUNK // 16)
            c = (i % (CHUNK // 16)) * 16
            vals = plsc.load_gather(tab_v, [src_v[r, pl.ds(c, 16)]])
            plsc.addupdate_scatter(acc_v, [dst_v[r, pl.ds(c, 16)]], vals)
            return carry

        lax.fori_loop(0, CPW * (CHUNK // 16), ebody, 0)
        pltpu.sync_copy(acc_v, out_hbm.at[wid])

    return scal_kernel(tab, srcr, dstr)


def _tc_prep(degp, x, w1):
    def body(degp_ref, x_ref, w1_ref, d1_ref, d2_ref, hs1_ref):
        deg = jnp.sum(degp_ref[...], axis=0) + 1.0
        dinv = lax.rsqrt(deg)
        d1 = dinv[0]
        d2 = dinv[1]
        d1_ref[...] = d1
        d2_ref[...] = d2
        h0 = jnp.dot(x_ref[...], w1_ref[...],
                     preferred_element_type=jnp.float32, precision=_HI)
        hs1_ref[...] = h0 * d1[0:N][:, None]

    return pl.pallas_call(
        body,
        out_shape=(
            jax.ShapeDtypeStruct((NROW,), jnp.float32),
            jax.ShapeDtypeStruct((NROW,), jnp.float32),
            jax.ShapeDtypeStruct((N, NH), jnp.float32),
        ),
    )(degp, x, w1)


def _tc_big_matmul(s, w2, dinv2col):
    MB = 200
    nm = N // MB

    def body(s_ref, w_ref, d_ref, o_ref):
        o_ref[...] = jnp.dot(s_ref[...], w_ref[...],
                             preferred_element_type=jnp.float32) * d_ref[...]

    return pl.pallas_call(
        body,
        grid=(nm,),
        in_specs=[
            pl.BlockSpec((MB, N), lambda i: (i, 0)),
            pl.BlockSpec((N, NH), lambda i: (0, 0)),
            pl.BlockSpec((MB, 1), lambda i: (i, 0)),
        ],
        out_specs=pl.BlockSpec((MB, NH), lambda i: (i, 0)),
        out_shape=jax.ShapeDtypeStruct((N, NH), jnp.float32),
    )(s, w2, dinv2col)


def _tc_combine(aggp, hs, dinv, b, wsc_row):
    def body(aggp_ref, hs_ref, d_ref, b_ref, wsc_ref, h_ref, ps_ref):
        a = aggp_ref[...]
        agg = a[0] + a[1]
        d = d_ref[...]
        base = agg[0:N] + hs_ref[...]
        hval = jnp.maximum(base * d[0:N][:, None] + b_ref[...][None, :], 0.0)
        hfull = jnp.concatenate(
            [hval, jnp.zeros((NROW - N, NH), jnp.float32)], axis=0)
        h_ref[...] = hfull
        p = jnp.sum(hfull * wsc_ref[...], axis=1)
        ps_ref[...] = d * p

    return pl.pallas_call(
        body,
        out_shape=(
            jax.ShapeDtypeStruct((NROW, NH), jnp.float32),
            jax.ShapeDtypeStruct((NROW,), jnp.float32),
        ),
    )(aggp, hs, dinv, b, wsc_row)


def _tc_readout(sagp, dinv, ps, bsc, h):
    def body(sagp_ref, d_ref, ps_ref, bsc_ref, h_ref, out_ref):
        sagg = jnp.sum(sagp_ref[...], axis=0)
        d = d_ref[...]
        score = d * (sagg + ps_ref[...]) + bsc_ref[0]

        bits = lax.bitcast_convert_type(score, jnp.uint32)
        key = jnp.where(bits >> 31 != 0, ~bits,
                        bits | jnp.uint32(0x80000000))
        idx = lax.broadcasted_iota(jnp.int32, (NROW,), 0)
        key = jnp.where(idx < N, key, jnp.uint32(0))

        t = jnp.uint32(0)
        for bit in range(31, -1, -1):
            cand = t | jnp.uint32(1 << bit)
            cnt = jnp.sum((key >= cand).astype(jnp.int32))
            t = jnp.where(cnt >= KTOP, cand, t)
        c_gt = jnp.sum((key > t).astype(jnp.int32))

        ties = (key == t)
        jcut = jnp.int32(0)
        for bit in range(13, -1, -1):
            cand = jcut + jnp.int32(1 << bit)
            f = c_gt + jnp.sum((ties & (idx < cand)).astype(jnp.int32))
            jcut = jnp.where(f < KTOP, cand, jcut)
        mask = (key > t) | (ties & (idx < jcut + 1))

        g = jnp.tanh(score)
        gated = h_ref[...] * g[:, None]
        mcol = mask.astype(jnp.float32)[:, None]
        xmax = jnp.max(jnp.where(mcol > 0.0, gated, -jnp.inf), axis=0)
        xsum = jnp.sum(gated * mcol, axis=0)
        out_ref[0, :] = xmax
        out_ref[1, :] = xsum / KTOP

    return pl.pallas_call(
        body,
        out_shape=jax.ShapeDtypeStruct((2, NH), jnp.float32),
    )(sagp, dinv, ps, bsc, h)


def _tc_head(x1p, x2p, lin1w, lin1b, lin3w, lin3b):
    def body(x1_ref, x2_ref, w1_ref, b1_ref, w3_ref, b3_ref, out_ref):
        z = jnp.concatenate(
            [x1_ref[0], x1_ref[1], x2_ref[0], x2_ref[1]], axis=0)
        zm = jnp.sum(z[:, None] * w1_ref[...], axis=0) + b1_ref[...]
        zm = jnp.maximum(zm, 0.0)
        o = jnp.sum(zm[:, None] * w3_ref[...], axis=0) + b3_ref[...]
        m = jnp.max(o)
        e = o - m
        out_ref[...] = (e - jnp.log(jnp.sum(jnp.exp(e))))[None, :]

    return pl.pallas_call(
        body,
        out_shape=jax.ShapeDtypeStruct((1, NCLS), jnp.float32),
    )(x1p, x2p, lin1w, lin1b, lin3w, lin3b)


def _rechunk(er):
    return er.reshape(EPAD // RCHUNK, RCHUNK)


def _pad_edges(ei):
    pad = EPAD - E
    pad_dst = N + jnp.arange(pad, dtype=jnp.int32) % (NROW - N)
    srcp = jnp.concatenate(
        [ei[0], jnp.zeros((pad,), jnp.int32)]).reshape(ECH, CHUNK)
    dstp = jnp.concatenate([ei[1], pad_dst]).reshape(ECH, CHUNK)
    return srcp, dstp


def kernel(x, s, edge1_index, edge2_index, batch, W1, b1, Wsc1, bsc1,
           W2, b2, Wsc2, bsc2, lin1_W, lin1_b, lin3_W, lin3_b):
    src1r, dst1r = _pad_edges(edge1_index)
    src2r, dst2r = _pad_edges(edge2_index)

    degp = _sc_degrees(dst1r, dst2r)
    dinv1, dinv2, hs1 = _tc_prep(degp, x, W1)
    hs2 = _tc_big_matmul(s, W2, dinv2.reshape(NROW, 1))

    aggp1 = _sc_row_agg(hs1, _rechunk(src1r), _rechunk(dst1r))
    h1, ps1 = _tc_combine(aggp1, hs1, dinv1, b1, Wsc1.reshape(1, NH))
    sagp1 = _sc_scalar_agg(ps1, src1r, dst1r)
    x1p = _tc_readout(sagp1, dinv1, ps1, bsc1, h1)

    aggp2 = _sc_row_agg(hs2, _rechunk(src2r), _rechunk(dst2r))
    h2, ps2 = _tc_combine(aggp2, hs2, dinv2, b2, Wsc2.reshape(1, NH))
    sagp2 = _sc_scalar_agg(ps2, src2r, dst2r)
    x2p = _tc_readout(sagp2, dinv2, ps2, bsc2, h2)

    return _tc_head(x1p, x2p, lin1_W, lin1_b, lin3_W, lin3_b)


# speedup vs baseline: 16.0192x; 1.0005x over previous
import functools

import jax
import jax.numpy as jnp
from jax import lax
from jax.experimental import pallas as pl
from jax.experimental.pallas import tpu as pltpu
from jax.experimental.pallas import tpu_sc as plsc

N = 10000
D = 128
NH = 128
E = 320000
NCLS = 10
KTOP = 5000

NCORES = 2
NSUB = 16
NW = NCORES * NSUB
CHUNK = 128
CPW = 80
EPAD = NW * CPW * CHUNK
ECH = EPAD // CHUNK
NROW = 10240
DUMMY = 10016
RPS = NROW // NSUB
NBUF = 4
RCHUNK = 64
RCPW = EPAD // RCHUNK // NW

_HI = lax.Precision.HIGHEST


def _mesh():
    return plsc.VectorSubcoreMesh(core_axis_name="c", subcore_axis_name="s")


_SC_PARAMS = pltpu.CompilerParams(use_tc_tiling_on_sc=False,
                                  needs_layout_passes=False)


def _sc_degrees(dst1r, dst2r):
    @functools.partial(
        pl.kernel,
        out_type=jax.ShapeDtypeStruct((NW, 2, NROW), jnp.float32),
        mesh=_mesh(),
        compiler_params=_SC_PARAMS,
        scratch_types=[
            pltpu.VMEM((CPW, CHUNK), jnp.int32),
            pltpu.VMEM((CPW, CHUNK), jnp.int32),
            pltpu.VMEM((NROW,), jnp.float32),
            pltpu.VMEM((NROW,), jnp.float32),
        ],
    )
    def deg_kernel(d1_hbm, d2_hbm, out_hbm, d1_v, d2_v, a1_v, a2_v):
        cid = lax.axis_index("c")
        sid = lax.axis_index("s")
        wid = sid * NCORES + cid
        base = wid * CPW
        pltpu.sync_copy(d1_hbm.at[pl.ds(base, CPW)], d1_v)
        pltpu.sync_copy(d2_hbm.at[pl.ds(base, CPW)], d2_v)
        z16 = jnp.zeros((16,), jnp.float32)
        ones = jnp.ones((16,), jnp.float32)

        def zbody(i, carry):
            a1_v[pl.ds(i * 16, 16)] = z16
            a2_v[pl.ds(i * 16, 16)] = z16
            return carry

        lax.fori_loop(0, NROW // 16, zbody, 0)

        def ebody(i, carry):
            r = i // (CHUNK // 16)
            c = (i % (CHUNK // 16)) * 16
            plsc.addupdate_scatter(a1_v, [d1_v[r, pl.ds(c, 16)]], ones)
            plsc.addupdate_scatter(a2_v, [d2_v[r, pl.ds(c, 16)]], ones)
            return carry

        lax.fori_loop(0, CPW * (CHUNK // 16), ebody, 0)
        pltpu.sync_copy(a1_v, out_hbm.at[wid, 0])
        pltpu.sync_copy(a2_v, out_hbm.at[wid, 1])

    return deg_kernel(dst1r, dst2r)


def _sc_row_agg(hs, srcr, dstr):
    @functools.partial(
        pl.kernel,
        out_type=jax.ShapeDtypeStruct((NCORES, NROW, NH), jnp.float32),
        mesh=_mesh(),
        compiler_params=_SC_PARAMS,
        scratch_types=[
            pltpu.VMEM((RCPW, RCHUNK), jnp.int32),
            pltpu.VMEM((NBUF, RCHUNK), jnp.int32),
            pltpu.VMEM((NBUF, RCHUNK, NH), jnp.float32),
            pltpu.VMEM_SHARED((NROW, NH), jnp.float32),
            [pltpu.SemaphoreType.DMA] * NBUF,
            [pltpu.SemaphoreType.DMA] * NBUF,
        ],
    )
    def rowagg_kernel(hs_hbm, src_hbm, dst_hbm, out_hbm,
                      src_v, dstr_v, rows_v, acc_sh, gsems, dsems):
        cid = lax.axis_index("c")
        sid = lax.axis_index("s")
        wid = sid * NCORES + cid
        base = wid * RCPW
        pltpu.sync_copy(src_hbm.at[pl.ds(base, RCPW)], src_v)
        z16 = jnp.zeros((16,), jnp.float32)

        def zb_body(i, carry):
            rows_v[0, i // 8, pl.ds((i % 8) * 16, 16)] = z16
            return carry

        lax.fori_loop(0, RCHUNK * (NH // 16), zb_body, 0)
        row0 = sid * RPS

        def zacc_body(t, carry):
            pltpu.sync_copy(rows_v.at[0],
                            acc_sh.at[pl.ds(row0 + t * RCHUNK, RCHUNK)])
            return carry

        lax.fori_loop(0, RPS // RCHUNK, zacc_body, 0)
        plsc.subcore_barrier()

        def start_chunk(j, b):
            pltpu.async_copy(dst_hbm.at[base + j], dstr_v.at[b], dsems[b])
            pltpu.async_copy(hs_hbm.at[src_v.at[j]], rows_v.at[b], gsems[b])

        def wait_chunk(b):
            pltpu.make_async_copy(dst_hbm.at[0], dstr_v.at[b],
                                  dsems[b]).wait()
            pltpu.make_async_copy(hs_hbm.at[pl.ds(0, RCHUNK)],
                                  rows_v.at[b], gsems[b]).wait()

        for b in range(NBUF):
            start_chunk(b, b)

        def ebody(g, carry):
            for b in range(NBUF):
                j = g * NBUF + b
                wait_chunk(b)
                pltpu.sync_copy(rows_v.at[b], acc_sh.at[dstr_v.at[b]],
                                add=True)
                jn = j + NBUF

                @pl.when(jn < RCPW)
                def _():
                    start_chunk(jn, b)

            return carry

        lax.fori_loop(0, RCPW // NBUF, ebody, 0)
        plsc.subcore_barrier()
        pltpu.sync_copy(acc_sh.at[pl.ds(row0, RPS)],
                        out_hbm.at[cid, pl.ds(row0, RPS)])

    return rowagg_kernel(hs, srcr, dstr)


def _sc_scalar_agg(tab, srcr, dstr):
    @functools.partial(
        pl.kernel,
        out_type=jax.ShapeDtypeStruct((NW, NROW), jnp.float32),
        mesh=_mesh(),
        compiler_params=_SC_PARAMS,
        scratch_types=[
            pltpu.VMEM((CPW, CHUNK), jnp.int32),
            pltpu.VMEM((CPW, CHUNK), jnp.int32),
            pltpu.VMEM((NROW,), jnp.float32),
            pltpu.VMEM((NROW,), jnp.float32),
        ],
    )
    def scal_kernel(tab_hbm, src_hbm, dst_hbm, out_hbm,
                    src_v, dst_v, tab_v, acc_v):
        cid = lax.axis_index("c")
        sid = lax.axis_index("s")
        wid = sid * NCORES + cid
        base = wid * CPW
        pltpu.sync_copy(src_hbm.at[pl.ds(base, CPW)], src_v)
        pltpu.sync_copy(dst_hbm.at[pl.ds(base, CPW)], dst_v)
        pltpu.sync_copy(tab_hbm, tab_v)
        z16 = jnp.zeros((16,), jnp.float32)

        def zbody(i, carry):
            acc_v[pl.ds(i * 16, 16)] = z16
            return carry

        lax.fori_loop(0, NROW // 16, zbody, 0)

        def ebody(i, carry):
            r = i // (CHUNK // 16)
            c = (i % (CHUNK // 16)) * 16
            vals = plsc.load_gather(tab_v, [src_v[r, pl.ds(c, 16)]])
            plsc.addupdate_scatter(acc_v, [dst_v[r, pl.ds(c, 16)]], vals)
            return carry

        lax.fori_loop(0, CPW * (CHUNK // 16), ebody, 0)
        pltpu.sync_copy(acc_v, out_hbm.at[wid])

    return scal_kernel(tab, srcr, dstr)


def _tc_prep(degp, x, w1):
    def body(degp_ref, x_ref, w1_ref, d1_ref, d2_ref, hs1_ref):
        deg = jnp.sum(degp_ref[...], axis=0) + 1.0
        dinv = lax.rsqrt(deg)
        d1 = dinv[0]
        d2 = dinv[1]
        d1_ref[...] = d1
        d2_ref[...] = d2
        h0 = jnp.dot(x_ref[...], w1_ref[...],
                     preferred_element_type=jnp.float32, precision=_HI)
        hs1_ref[...] = h0 * d1[0:N][:, None]

    return pl.pallas_call(
        body,
        out_shape=(
            jax.ShapeDtypeStruct((NROW,), jnp.float32),
            jax.ShapeDtypeStruct((NROW,), jnp.float32),
            jax.ShapeDtypeStruct((N, NH), jnp.float32),
        ),
    )(degp, x, w1)


def _tc_big_matmul(s, w2, dinv2col):
    MB = 200
    nm = N // MB

    def body(s_ref, w_ref, d_ref, o_ref):
        o_ref[...] = jnp.dot(s_ref[...], w_ref[...],
                             preferred_element_type=jnp.float32) * d_ref[...]

    return pl.pallas_call(
        body,
        grid=(nm,),
        in_specs=[
            pl.BlockSpec((MB, N), lambda i: (i, 0)),
            pl.BlockSpec((N, NH), lambda i: (0, 0)),
            pl.BlockSpec((MB, 1), lambda i: (i, 0)),
        ],
        out_specs=pl.BlockSpec((MB, NH), lambda i: (i, 0)),
        out_shape=jax.ShapeDtypeStruct((N, NH), jnp.float32),
    )(s, w2, dinv2col)


def _tc_combine(aggp, hs, dinv, b, wsc_row):
    def body(aggp_ref, hs_ref, d_ref, b_ref, wsc_ref, h_ref, ps_ref):
        a = aggp_ref[...]
        agg = a[0] + a[1]
        d = d_ref[...]
        base = agg[0:N] + hs_ref[...]
        hval = jnp.maximum(base * d[0:N][:, None] + b_ref[...][None, :], 0.0)
        hfull = jnp.concatenate(
            [hval, jnp.zeros((NROW - N, NH), jnp.float32)], axis=0)
        h_ref[...] = hfull
        p = jnp.sum(hfull * wsc_ref[...], axis=1)
        ps_ref[...] = d * p

    return pl.pallas_call(
        body,
        out_shape=(
            jax.ShapeDtypeStruct((NROW, NH), jnp.float32),
            jax.ShapeDtypeStruct((NROW,), jnp.float32),
        ),
    )(aggp, hs, dinv, b, wsc_row)


def _readout_core(sagp_ref, d_ref, ps_ref, bsc_ref, h_ref):
    sagg = jnp.sum(sagp_ref[...], axis=0)
    d = d_ref[...]
    score = d * (sagg + ps_ref[...]) + bsc_ref[0]

    bits = lax.bitcast_convert_type(score, jnp.uint32)
    key = jnp.where(bits >> 31 != 0, ~bits,
                    bits | jnp.uint32(0x80000000))
    idx = lax.broadcasted_iota(jnp.int32, (NROW,), 0)
    key = jnp.where(idx < N, key, jnp.uint32(0))

    t = jnp.uint32(0)
    for bit in range(31, -1, -1):
        cand = t | jnp.uint32(1 << bit)
        cnt = jnp.sum((key >= cand).astype(jnp.int32))
        t = jnp.where(cnt >= KTOP, cand, t)
    c_gt = jnp.sum((key > t).astype(jnp.int32))

    ties = (key == t)
    jcut = jnp.int32(0)
    for bit in range(13, -1, -1):
        cand = jcut + jnp.int32(1 << bit)
        f = c_gt + jnp.sum((ties & (idx < cand)).astype(jnp.int32))
        jcut = jnp.where(f < KTOP, cand, jcut)
    mask = (key > t) | (ties & (idx < jcut + 1))

    g = jnp.tanh(score)
    gated = h_ref[...] * g[:, None]
    mcol = mask.astype(jnp.float32)[:, None]
    xmax = jnp.max(jnp.where(mcol > 0.0, gated, -jnp.inf), axis=0)
    xsum = jnp.sum(gated * mcol, axis=0)
    return xmax, xsum / KTOP


def _tc_readout(sagp, dinv, ps, bsc, h):
    def body(sagp_ref, d_ref, ps_ref, bsc_ref, h_ref, out_ref):
        xmax, xmean = _readout_core(sagp_ref, d_ref, ps_ref, bsc_ref, h_ref)
        out_ref[0, :] = xmax
        out_ref[1, :] = xmean

    return pl.pallas_call(
        body,
        out_shape=jax.ShapeDtypeStruct((2, NH), jnp.float32),
    )(sagp, dinv, ps, bsc, h)


def _tc_readout_head(sagp, dinv, ps, bsc, h, x1p, lin1w, lin1b, lin3w, lin3b):
    def body(sagp_ref, d_ref, ps_ref, bsc_ref, h_ref, x1_ref,
             w1_ref, b1_ref, w3_ref, b3_ref, out_ref):
        xmax, xmean = _readout_core(sagp_ref, d_ref, ps_ref, bsc_ref, h_ref)
        z = jnp.concatenate([x1_ref[0], x1_ref[1], xmax, xmean], axis=0)
        zm = jnp.sum(z[:, None] * w1_ref[...], axis=0) + b1_ref[...]
        zm = jnp.maximum(zm, 0.0)
        o = jnp.sum(zm[:, None] * w3_ref[...], axis=0) + b3_ref[...]
        m = jnp.max(o)
        e = o - m
        out_ref[...] = (e - jnp.log(jnp.sum(jnp.exp(e))))[None, :]

    return pl.pallas_call(
        body,
        out_shape=jax.ShapeDtypeStruct((1, NCLS), jnp.float32),
    )(sagp, dinv, ps, bsc, h, x1p, lin1w, lin1b, lin3w, lin3b)


def _tc_head(x1p, x2p, lin1w, lin1b, lin3w, lin3b):
    def body(x1_ref, x2_ref, w1_ref, b1_ref, w3_ref, b3_ref, out_ref):
        z = jnp.concatenate(
            [x1_ref[0], x1_ref[1], x2_ref[0], x2_ref[1]], axis=0)
        zm = jnp.sum(z[:, None] * w1_ref[...], axis=0) + b1_ref[...]
        zm = jnp.maximum(zm, 0.0)
        o = jnp.sum(zm[:, None] * w3_ref[...], axis=0) + b3_ref[...]
        m = jnp.max(o)
        e = o - m
        out_ref[...] = (e - jnp.log(jnp.sum(jnp.exp(e))))[None, :]

    return pl.pallas_call(
        body,
        out_shape=jax.ShapeDtypeStruct((1, NCLS), jnp.float32),
    )(x1p, x2p, lin1w, lin1b, lin3w, lin3b)


def _rechunk(er):
    return er.reshape(EPAD // RCHUNK, RCHUNK)


def _pad_edges(ei):
    pad = EPAD - E
    pad_dst = N + jnp.arange(pad, dtype=jnp.int32) % (NROW - N)
    srcp = jnp.concatenate(
        [ei[0], jnp.zeros((pad,), jnp.int32)]).reshape(ECH, CHUNK)
    dstp = jnp.concatenate([ei[1], pad_dst]).reshape(ECH, CHUNK)
    return srcp, dstp


def kernel(x, s, edge1_index, edge2_index, batch, W1, b1, Wsc1, bsc1,
           W2, b2, Wsc2, bsc2, lin1_W, lin1_b, lin3_W, lin3_b):
    src1r, dst1r = _pad_edges(edge1_index)
    src2r, dst2r = _pad_edges(edge2_index)

    degp = _sc_degrees(dst1r, dst2r)
    dinv1, dinv2, hs1 = _tc_prep(degp, x, W1)
    hs2 = _tc_big_matmul(s, W2, dinv2.reshape(NROW, 1))

    aggp1 = _sc_row_agg(hs1, _rechunk(src1r), _rechunk(dst1r))
    h1, ps1 = _tc_combine(aggp1, hs1, dinv1, b1, Wsc1.reshape(1, NH))
    sagp1 = _sc_scalar_agg(ps1, src1r, dst1r)
    x1p = _tc_readout(sagp1, dinv1, ps1, bsc1, h1)

    aggp2 = _sc_row_agg(hs2, _rechunk(src2r), _rechunk(dst2r))
    h2, ps2 = _tc_combine(aggp2, hs2, dinv2, b2, Wsc2.reshape(1, NH))
    sagp2 = _sc_scalar_agg(ps2, src2r, dst2r)
    return _tc_readout_head(sagp2, dinv2, ps2, bsc2, h2, x1p,
                            lin1_W, lin1_b, lin3_W, lin3_b)


# speedup vs baseline: 16.0297x; 1.0007x over previous
import functools

import jax
import jax.numpy as jnp
from jax import lax
from jax.experimental import pallas as pl
from jax.experimental.pallas import tpu as pltpu
from jax.experimental.pallas import tpu_sc as plsc

N = 10000
D = 128
NH = 128
E = 320000
NCLS = 10
KTOP = 5000

NCORES = 2
NSUB = 16
NW = NCORES * NSUB
CHUNK = 128
CPW = 80
EPAD = NW * CPW * CHUNK
ECH = EPAD // CHUNK
NROW = 10240
RPS = NROW // NSUB
NBUF = 4
RCHUNK = 64
RCPW = EPAD // RCHUNK // NW

_HI = lax.Precision.HIGHEST


def _mesh():
    return plsc.VectorSubcoreMesh(core_axis_name="c", subcore_axis_name="s")


_SC_PARAMS = pltpu.CompilerParams(use_tc_tiling_on_sc=False,
                                  needs_layout_passes=False)


def _sc_degrees(dst1r, dst2r):
    @functools.partial(
        pl.kernel,
        out_type=jax.ShapeDtypeStruct((NW, 2, NROW), jnp.float32),
        mesh=_mesh(),
        compiler_params=_SC_PARAMS,
        scratch_types=[
            pltpu.VMEM((CPW, CHUNK), jnp.int32),
            pltpu.VMEM((CPW, CHUNK), jnp.int32),
            pltpu.VMEM((NROW,), jnp.float32),
            pltpu.VMEM((NROW,), jnp.float32),
        ],
    )
    def deg_kernel(d1_hbm, d2_hbm, out_hbm, d1_v, d2_v, a1_v, a2_v):
        cid = lax.axis_index("c")
        sid = lax.axis_index("s")
        wid = sid * NCORES + cid
        base = wid * CPW
        pltpu.sync_copy(d1_hbm.at[pl.ds(base, CPW)], d1_v)
        pltpu.sync_copy(d2_hbm.at[pl.ds(base, CPW)], d2_v)
        z16 = jnp.zeros((16,), jnp.float32)
        ones = jnp.ones((16,), jnp.float32)

        def zbody(i, carry):
            a1_v[pl.ds(i * 16, 16)] = z16
            a2_v[pl.ds(i * 16, 16)] = z16
            return carry

        lax.fori_loop(0, NROW // 16, zbody, 0)

        def ebody(i, carry):
            r = i // (CHUNK // 16)
            c = (i % (CHUNK // 16)) * 16
            plsc.addupdate_scatter(a1_v, [d1_v[r, pl.ds(c, 16)]], ones)
            plsc.addupdate_scatter(a2_v, [d2_v[r, pl.ds(c, 16)]], ones)
            return carry

        lax.fori_loop(0, CPW * (CHUNK // 16), ebody, 0)
        pltpu.sync_copy(a1_v, out_hbm.at[wid, 0])
        pltpu.sync_copy(a2_v, out_hbm.at[wid, 1])

    return deg_kernel(dst1r, dst2r)


def _sc_row_agg(hs, srcr, dstr):
    @functools.partial(
        pl.kernel,
        out_type=jax.ShapeDtypeStruct((NCORES, NROW, NH), jnp.float32),
        mesh=_mesh(),
        compiler_params=_SC_PARAMS,
        scratch_types=[
            pltpu.VMEM((RCPW, RCHUNK), jnp.int32),
            pltpu.VMEM((NBUF, RCHUNK), jnp.int32),
            pltpu.VMEM((NBUF, RCHUNK, NH), jnp.float32),
            pltpu.VMEM_SHARED((NROW, NH), jnp.float32),
            [pltpu.SemaphoreType.DMA] * NBUF,
            [pltpu.SemaphoreType.DMA] * NBUF,
        ],
    )
    def rowagg_kernel(hs_hbm, src_hbm, dst_hbm, out_hbm,
                      src_v, dstr_v, rows_v, acc_sh, gsems, dsems):
        cid = lax.axis_index("c")
        sid = lax.axis_index("s")
        wid = sid * NCORES + cid
        base = wid * RCPW
        pltpu.sync_copy(src_hbm.at[pl.ds(base, RCPW)], src_v)
        z16 = jnp.zeros((16,), jnp.float32)

        def zb_body(i, carry):
            rows_v[0, i // 8, pl.ds((i % 8) * 16, 16)] = z16
            return carry

        lax.fori_loop(0, RCHUNK * (NH // 16), zb_body, 0)
        row0 = sid * RPS

        def zacc_body(t, carry):
            pltpu.sync_copy(rows_v.at[0],
                            acc_sh.at[pl.ds(row0 + t * RCHUNK, RCHUNK)])
            return carry

        lax.fori_loop(0, RPS // RCHUNK, zacc_body, 0)
        plsc.subcore_barrier()

        def start_chunk(j, b):
            pltpu.async_copy(dst_hbm.at[base + j], dstr_v.at[b], dsems[b])
            pltpu.async_copy(hs_hbm.at[src_v.at[j]], rows_v.at[b], gsems[b])

        def wait_chunk(b):
            pltpu.make_async_copy(dst_hbm.at[0], dstr_v.at[b],
                                  dsems[b]).wait()
            pltpu.make_async_copy(hs_hbm.at[pl.ds(0, RCHUNK)],
                                  rows_v.at[b], gsems[b]).wait()

        for b in range(NBUF):
            start_chunk(b, b)

        def ebody(g, carry):
            for b in range(NBUF):
                j = g * NBUF + b
                wait_chunk(b)
                pltpu.sync_copy(rows_v.at[b], acc_sh.at[dstr_v.at[b]],
                                add=True)
                jn = j + NBUF

                @pl.when(jn < RCPW)
                def _():
                    start_chunk(jn, b)

            return carry

        lax.fori_loop(0, RCPW // NBUF, ebody, 0)
        plsc.subcore_barrier()
        pltpu.sync_copy(acc_sh.at[pl.ds(row0, RPS)],
                        out_hbm.at[cid, pl.ds(row0, RPS)])

    return rowagg_kernel(hs, srcr, dstr)


def _sc_scalar_agg(tab, srcr, dstr):
    @functools.partial(
        pl.kernel,
        out_type=jax.ShapeDtypeStruct((NW, NROW), jnp.float32),
        mesh=_mesh(),
        compiler_params=_SC_PARAMS,
        scratch_types=[
            pltpu.VMEM((CPW, CHUNK), jnp.int32),
            pltpu.VMEM((CPW, CHUNK), jnp.int32),
            pltpu.VMEM((NROW,), jnp.float32),
            pltpu.VMEM((NROW,), jnp.float32),
        ],
    )
    def scal_kernel(tab_hbm, src_hbm, dst_hbm, out_hbm,
                    src_v, dst_v, tab_v, acc_v):
        cid = lax.axis_index("c")
        sid = lax.axis_index("s")
        wid = sid * NCORES + cid
        base = wid * CPW
        pltpu.sync_copy(src_hbm.at[pl.ds(base, CPW)], src_v)
        pltpu.sync_copy(dst_hbm.at[pl.ds(base, CPW)], dst_v)
        pltpu.sync_copy(tab_hbm, tab_v)
        z16 = jnp.zeros((16,), jnp.float32)

        def zbody(i, carry):
            acc_v[pl.ds(i * 16, 16)] = z16
            return carry

        lax.fori_loop(0, NROW // 16, zbody, 0)

        def ebody(i, carry):
            r = i // (CHUNK // 16)
            c = (i % (CHUNK // 16)) * 16
            vals = plsc.load_gather(tab_v, [src_v[r, pl.ds(c, 16)]])
            plsc.addupdate_scatter(acc_v, [dst_v[r, pl.ds(c, 16)]], vals)
            return carry

        lax.fori_loop(0, CPW * (CHUNK // 16), ebody, 0)
        pltpu.sync_copy(acc_v, out_hbm.at[wid])

    return scal_kernel(tab, srcr, dstr)


def _tc_prep(degp, x, w1):
    def body(degp_ref, x_ref, w1_ref, d1_ref, d2_ref, hs1_ref):
        deg = jnp.sum(degp_ref[...], axis=0) + 1.0
        dinv = lax.rsqrt(deg)
        d1 = dinv[0]
        d2 = dinv[1]
        d1_ref[...] = d1
        d2_ref[...] = d2
        h0 = jnp.dot(x_ref[...], w1_ref[...],
                     preferred_element_type=jnp.float32, precision=_HI)
        hs1_ref[...] = h0 * d1[0:N][:, None]

    return pl.pallas_call(
        body,
        out_shape=(
            jax.ShapeDtypeStruct((NROW,), jnp.float32),
            jax.ShapeDtypeStruct((NROW,), jnp.float32),
            jax.ShapeDtypeStruct((N, NH), jnp.float32),
        ),
    )(degp, x, w1)


def _tc_big_matmul(s, w2, dinv2col):
    MB = 200
    nm = N // MB

    def body(s_ref, w_ref, d_ref, o_ref):
        o_ref[...] = jnp.dot(s_ref[...], w_ref[...],
                             preferred_element_type=jnp.float32) * d_ref[...]

    return pl.pallas_call(
        body,
        grid=(nm,),
        in_specs=[
            pl.BlockSpec((MB, N), lambda i: (i, 0)),
            pl.BlockSpec((N, NH), lambda i: (0, 0)),
            pl.BlockSpec((MB, 1), lambda i: (i, 0)),
        ],
        out_specs=pl.BlockSpec((MB, NH), lambda i: (i, 0)),
        out_shape=jax.ShapeDtypeStruct((N, NH), jnp.float32),
    )(s, w2, dinv2col)


def _tc_combine(aggp, hs, dinv, b, wsc_row):
    def body(aggp_ref, hs_ref, d_ref, b_ref, wsc_ref, h_ref, ps_ref):
        a = aggp_ref[...]
        agg = a[0] + a[1]
        d = d_ref[...]
        base = agg[0:N] + hs_ref[...]
        hval = jnp.maximum(base * d[0:N][:, None] + b_ref[...][None, :], 0.0)
        hfull = jnp.concatenate(
            [hval, jnp.zeros((NROW - N, NH), jnp.float32)], axis=0)
        h_ref[...] = hfull
        p = jnp.sum(hfull * wsc_ref[...], axis=1)
        ps_ref[...] = d * p

    return pl.pallas_call(
        body,
        out_shape=(
            jax.ShapeDtypeStruct((NROW, NH), jnp.float32),
            jax.ShapeDtypeStruct((NROW,), jnp.float32),
        ),
    )(aggp, hs, dinv, b, wsc_row)


def _readout_core(sagp_ref, d_ref, ps_ref, bsc_ref, h_ref):
    sagg = jnp.sum(sagp_ref[...], axis=0)
    d = d_ref[...]
    score = d * (sagg + ps_ref[...]) + bsc_ref[0]

    bits = lax.bitcast_convert_type(score, jnp.uint32)
    key = jnp.where(bits >> 31 != 0, ~bits,
                    bits | jnp.uint32(0x80000000))
    idx = lax.broadcasted_iota(jnp.int32, (NROW,), 0)
    key = jnp.where(idx < N, key, jnp.uint32(0))

    t = jnp.uint32(0)
    for bit in range(31, -1, -1):
        cand = t | jnp.uint32(1 << bit)
        cnt = jnp.sum((key >= cand).astype(jnp.int32))
        t = jnp.where(cnt >= KTOP, cand, t)
    c_gt = jnp.sum((key > t).astype(jnp.int32))

    ties = (key == t)
    jcut = jnp.int32(0)
    for bit in range(13, -1, -1):
        cand = jcut + jnp.int32(1 << bit)
        f = c_gt + jnp.sum((ties & (idx < cand)).astype(jnp.int32))
        jcut = jnp.where(f < KTOP, cand, jcut)
    mask = (key > t) | (ties & (idx < jcut + 1))

    g = jnp.tanh(score)
    gated = h_ref[...] * g[:, None]
    mcol = mask.astype(jnp.float32)[:, None]
    xmax = jnp.max(jnp.where(mcol > 0.0, gated, -jnp.inf), axis=0)
    xsum = jnp.sum(gated * mcol, axis=0)
    return xmax, xsum / KTOP


def _tc_readout(sagp, dinv, ps, bsc, h):
    def body(sagp_ref, d_ref, ps_ref, bsc_ref, h_ref, out_ref):
        xmax, xmean = _readout_core(sagp_ref, d_ref, ps_ref, bsc_ref, h_ref)
        out_ref[0, :] = xmax
        out_ref[1, :] = xmean

    return pl.pallas_call(
        body,
        out_shape=jax.ShapeDtypeStruct((2, NH), jnp.float32),
    )(sagp, dinv, ps, bsc, h)


def _tc_readout_head(sagp, dinv, ps, bsc, h, x1p, lin1w, lin1b, lin3w, lin3b):
    def body(sagp_ref, d_ref, ps_ref, bsc_ref, h_ref, x1_ref,
             w1_ref, b1_ref, w3_ref, b3_ref, out_ref):
        xmax, xmean = _readout_core(sagp_ref, d_ref, ps_ref, bsc_ref, h_ref)
        z = jnp.concatenate([x1_ref[0], x1_ref[1], xmax, xmean], axis=0)
        zm = jnp.sum(z[:, None] * w1_ref[...], axis=0) + b1_ref[...]
        zm = jnp.maximum(zm, 0.0)
        o = jnp.sum(zm[:, None] * w3_ref[...], axis=0) + b3_ref[...]
        m = jnp.max(o)
        e = o - m
        out_ref[...] = (e - jnp.log(jnp.sum(jnp.exp(e))))[None, :]

    return pl.pallas_call(
        body,
        out_shape=jax.ShapeDtypeStruct((1, NCLS), jnp.float32),
    )(sagp, dinv, ps, bsc, h, x1p, lin1w, lin1b, lin3w, lin3b)


def _rechunk(er):
    return er.reshape(EPAD // RCHUNK, RCHUNK)


def _pad_edges(ei):
    pad = EPAD - E
    pad_dst = N + jnp.arange(pad, dtype=jnp.int32) % (NROW - N)
    srcp = jnp.concatenate(
        [ei[0], jnp.zeros((pad,), jnp.int32)]).reshape(ECH, CHUNK)
    dstp = jnp.concatenate([ei[1], pad_dst]).reshape(ECH, CHUNK)
    return srcp, dstp


def kernel(x, s, edge1_index, edge2_index, batch, W1, b1, Wsc1, bsc1,
           W2, b2, Wsc2, bsc2, lin1_W, lin1_b, lin3_W, lin3_b):
    src1r, dst1r = _pad_edges(edge1_index)
    src2r, dst2r = _pad_edges(edge2_index)

    degp = _sc_degrees(dst1r, dst2r)
    dinv1, dinv2, hs1 = _tc_prep(degp, x, W1)
    hs2 = _tc_big_matmul(s, W2, dinv2.reshape(NROW, 1))

    aggp1 = _sc_row_agg(hs1, _rechunk(src1r), _rechunk(dst1r))
    h1, ps1 = _tc_combine(aggp1, hs1, dinv1, b1, Wsc1.reshape(1, NH))
    sagp1 = _sc_scalar_agg(ps1, src1r, dst1r)
    x1p = _tc_readout(sagp1, dinv1, ps1, bsc1, h1)

    aggp2 = _sc_row_agg(hs2, _rechunk(src2r), _rechunk(dst2r))
    h2, ps2 = _tc_combine(aggp2, hs2, dinv2, b2, Wsc2.reshape(1, NH))
    sagp2 = _sc_scalar_agg(ps2, src2r, dst2r)
    return _tc_readout_head(sagp2, dinv2, ps2, bsc2, h2, x1p,
                            lin1_W, lin1_b, lin3_W, lin3_b)


# speedup vs baseline: 16.0574x; 1.0017x over previous
import functools

import jax
import jax.numpy as jnp
from jax import lax
from jax.experimental import pallas as pl
from jax.experimental.pallas import tpu as pltpu
from jax.experimental.pallas import tpu_sc as plsc

N = 10000
D = 128
NH = 128
E = 320000
NCLS = 10
KTOP = 5000

NCORES = 2
NSUB = 16
NW = NCORES * NSUB
CHUNK = 128
CPW = 80
EPAD = NW * CPW * CHUNK
ECH = EPAD // CHUNK
NROW = 10240
RPS = NROW // NSUB
NBUF = 8
RCHUNK = 32
RCPW = EPAD // RCHUNK // NW

_HI = lax.Precision.HIGHEST


def _mesh():
    return plsc.VectorSubcoreMesh(core_axis_name="c", subcore_axis_name="s")


_SC_PARAMS = pltpu.CompilerParams(use_tc_tiling_on_sc=False,
                                  needs_layout_passes=False)


def _sc_degrees(dst1r, dst2r):
    @functools.partial(
        pl.kernel,
        out_type=jax.ShapeDtypeStruct((NW, 2, NROW), jnp.float32),
        mesh=_mesh(),
        compiler_params=_SC_PARAMS,
        scratch_types=[
            pltpu.VMEM((CPW, CHUNK), jnp.int32),
            pltpu.VMEM((CPW, CHUNK), jnp.int32),
            pltpu.VMEM((NROW,), jnp.float32),
            pltpu.VMEM((NROW,), jnp.float32),
        ],
    )
    def deg_kernel(d1_hbm, d2_hbm, out_hbm, d1_v, d2_v, a1_v, a2_v):
        cid = lax.axis_index("c")
        sid = lax.axis_index("s")
        wid = sid * NCORES + cid
        base = wid * CPW
        pltpu.sync_copy(d1_hbm.at[pl.ds(base, CPW)], d1_v)
        pltpu.sync_copy(d2_hbm.at[pl.ds(base, CPW)], d2_v)
        z16 = jnp.zeros((16,), jnp.float32)
        ones = jnp.ones((16,), jnp.float32)

        def zbody(i, carry):
            a1_v[pl.ds(i * 16, 16)] = z16
            a2_v[pl.ds(i * 16, 16)] = z16
            return carry

        lax.fori_loop(0, NROW // 16, zbody, 0)

        def ebody(i, carry):
            r = i // (CHUNK // 16)
            c = (i % (CHUNK // 16)) * 16
            plsc.addupdate_scatter(a1_v, [d1_v[r, pl.ds(c, 16)]], ones)
            plsc.addupdate_scatter(a2_v, [d2_v[r, pl.ds(c, 16)]], ones)
            return carry

        lax.fori_loop(0, CPW * (CHUNK // 16), ebody, 0)
        pltpu.sync_copy(a1_v, out_hbm.at[wid, 0])
        pltpu.sync_copy(a2_v, out_hbm.at[wid, 1])

    return deg_kernel(dst1r, dst2r)


def _sc_row_agg(hs, srcr, dstr):
    @functools.partial(
        pl.kernel,
        out_type=jax.ShapeDtypeStruct((NCORES, NROW, NH), jnp.float32),
        mesh=_mesh(),
        compiler_params=_SC_PARAMS,
        scratch_types=[
            pltpu.VMEM((RCPW, RCHUNK), jnp.int32),
            pltpu.VMEM((NBUF, RCHUNK), jnp.int32),
            pltpu.VMEM((NBUF, RCHUNK, NH), jnp.float32),
            pltpu.VMEM_SHARED((NROW, NH), jnp.float32),
            [pltpu.SemaphoreType.DMA] * NBUF,
            [pltpu.SemaphoreType.DMA] * NBUF,
        ],
    )
    def rowagg_kernel(hs_hbm, src_hbm, dst_hbm, out_hbm,
                      src_v, dstr_v, rows_v, acc_sh, gsems, dsems):
        cid = lax.axis_index("c")
        sid = lax.axis_index("s")
        wid = sid * NCORES + cid
        base = wid * RCPW
        pltpu.sync_copy(src_hbm.at[pl.ds(base, RCPW)], src_v)
        z16 = jnp.zeros((16,), jnp.float32)

        def zb_body(i, carry):
            rows_v[0, i // 8, pl.ds((i % 8) * 16, 16)] = z16
            return carry

        lax.fori_loop(0, RCHUNK * (NH // 16), zb_body, 0)
        row0 = sid * RPS

        def zacc_body(t, carry):
            pltpu.sync_copy(rows_v.at[0],
                            acc_sh.at[pl.ds(row0 + t * RCHUNK, RCHUNK)])
            return carry

        lax.fori_loop(0, RPS // RCHUNK, zacc_body, 0)
        plsc.subcore_barrier()

        def start_chunk(j, b):
            pltpu.async_copy(dst_hbm.at[base + j], dstr_v.at[b], dsems[b])
            pltpu.async_copy(hs_hbm.at[src_v.at[j]], rows_v.at[b], gsems[b])

        def wait_chunk(b):
            pltpu.make_async_copy(dst_hbm.at[0], dstr_v.at[b],
                                  dsems[b]).wait()
            pltpu.make_async_copy(hs_hbm.at[pl.ds(0, RCHUNK)],
                                  rows_v.at[b], gsems[b]).wait()

        for b in range(NBUF):
            start_chunk(b, b)

        def ebody(g, carry):
            for b in range(NBUF):
                j = g * NBUF + b
                wait_chunk(b)
                pltpu.sync_copy(rows_v.at[b], acc_sh.at[dstr_v.at[b]],
                                add=True)
                jn = j + NBUF

                @pl.when(jn < RCPW)
                def _():
                    start_chunk(jn, b)

            return carry

        lax.fori_loop(0, RCPW // NBUF, ebody, 0)
        plsc.subcore_barrier()
        pltpu.sync_copy(acc_sh.at[pl.ds(row0, RPS)],
                        out_hbm.at[cid, pl.ds(row0, RPS)])

    return rowagg_kernel(hs, srcr, dstr)


def _sc_scalar_agg(tab, srcr, dstr):
    @functools.partial(
        pl.kernel,
        out_type=jax.ShapeDtypeStruct((NW, NROW), jnp.float32),
        mesh=_mesh(),
        compiler_params=_SC_PARAMS,
        scratch_types=[
            pltpu.VMEM((CPW, CHUNK), jnp.int32),
            pltpu.VMEM((CPW, CHUNK), jnp.int32),
            pltpu.VMEM((NROW,), jnp.float32),
            pltpu.VMEM((NROW,), jnp.float32),
        ],
    )
    def scal_kernel(tab_hbm, src_hbm, dst_hbm, out_hbm,
                    src_v, dst_v, tab_v, acc_v):
        cid = lax.axis_index("c")
        sid = lax.axis_index("s")
        wid = sid * NCORES + cid
        base = wid * CPW
        pltpu.sync_copy(src_hbm.at[pl.ds(base, CPW)], src_v)
        pltpu.sync_copy(dst_hbm.at[pl.ds(base, CPW)], dst_v)
        pltpu.sync_copy(tab_hbm, tab_v)
        z16 = jnp.zeros((16,), jnp.float32)

        def zbody(i, carry):
            acc_v[pl.ds(i * 16, 16)] = z16
            return carry

        lax.fori_loop(0, NROW // 16, zbody, 0)

        def ebody(i, carry):
            r = i // (CHUNK // 16)
            c = (i % (CHUNK // 16)) * 16
            vals = plsc.load_gather(tab_v, [src_v[r, pl.ds(c, 16)]])
            plsc.addupdate_scatter(acc_v, [dst_v[r, pl.ds(c, 16)]], vals)
            return carry

        lax.fori_loop(0, CPW * (CHUNK // 16), ebody, 0)
        pltpu.sync_copy(acc_v, out_hbm.at[wid])

    return scal_kernel(tab, srcr, dstr)


def _tc_prep(degp, x, w1):
    def body(degp_ref, x_ref, w1_ref, d1_ref, d2_ref, hs1_ref):
        deg = jnp.sum(degp_ref[...], axis=0) + 1.0
        dinv = lax.rsqrt(deg)
        d1 = dinv[0]
        d2 = dinv[1]
        d1_ref[...] = d1
        d2_ref[...] = d2
        h0 = jnp.dot(x_ref[...], w1_ref[...],
                     preferred_element_type=jnp.float32, precision=_HI)
        hs1_ref[...] = h0 * d1[0:N][:, None]

    return pl.pallas_call(
        body,
        out_shape=(
            jax.ShapeDtypeStruct((NROW,), jnp.float32),
            jax.ShapeDtypeStruct((NROW,), jnp.float32),
            jax.ShapeDtypeStruct((N, NH), jnp.float32),
        ),
    )(degp, x, w1)


def _tc_big_matmul(s, w2, dinv2col):
    MB = 200
    nm = N // MB

    def body(s_ref, w_ref, d_ref, o_ref):
        o_ref[...] = jnp.dot(s_ref[...], w_ref[...],
                             preferred_element_type=jnp.float32) * d_ref[...]

    return pl.pallas_call(
        body,
        grid=(nm,),
        in_specs=[
            pl.BlockSpec((MB, N), lambda i: (i, 0)),
            pl.BlockSpec((N, NH), lambda i: (0, 0)),
            pl.BlockSpec((MB, 1), lambda i: (i, 0)),
        ],
        out_specs=pl.BlockSpec((MB, NH), lambda i: (i, 0)),
        out_shape=jax.ShapeDtypeStruct((N, NH), jnp.float32),
    )(s, w2, dinv2col)


def _tc_combine(aggp, hs, dinv, b, wsc_row):
    def body(aggp_ref, hs_ref, d_ref, b_ref, wsc_ref, h_ref, ps_ref):
        a = aggp_ref[...]
        agg = a[0] + a[1]
        d = d_ref[...]
        base = agg[0:N] + hs_ref[...]
        hval = jnp.maximum(base * d[0:N][:, None] + b_ref[...][None, :], 0.0)
        hfull = jnp.concatenate(
            [hval, jnp.zeros((NROW - N, NH), jnp.float32)], axis=0)
        h_ref[...] = hfull
        p = jnp.sum(hfull * wsc_ref[...], axis=1)
        ps_ref[...] = d * p

    return pl.pallas_call(
        body,
        out_shape=(
            jax.ShapeDtypeStruct((NROW, NH), jnp.float32),
            jax.ShapeDtypeStruct((NROW,), jnp.float32),
        ),
    )(aggp, hs, dinv, b, wsc_row)


def _readout_core(sagp_ref, d_ref, ps_ref, bsc_ref, h_ref):
    sagg = jnp.sum(sagp_ref[...], axis=0)
    d = d_ref[...]
    score = d * (sagg + ps_ref[...]) + bsc_ref[0]

    bits = lax.bitcast_convert_type(score, jnp.uint32)
    key = jnp.where(bits >> 31 != 0, ~bits,
                    bits | jnp.uint32(0x80000000))
    idx = lax.broadcasted_iota(jnp.int32, (NROW,), 0)
    key = jnp.where(idx < N, key, jnp.uint32(0))

    t = jnp.uint32(0)
    for bit in range(31, -1, -1):
        cand = t | jnp.uint32(1 << bit)
        cnt = jnp.sum((key >= cand).astype(jnp.int32))
        t = jnp.where(cnt >= KTOP, cand, t)
    c_gt = jnp.sum((key > t).astype(jnp.int32))

    ties = (key == t)
    jcut = jnp.int32(0)
    for bit in range(13, -1, -1):
        cand = jcut + jnp.int32(1 << bit)
        f = c_gt + jnp.sum((ties & (idx < cand)).astype(jnp.int32))
        jcut = jnp.where(f < KTOP, cand, jcut)
    mask = (key > t) | (ties & (idx < jcut + 1))

    g = jnp.tanh(score)
    gated = h_ref[...] * g[:, None]
    mcol = mask.astype(jnp.float32)[:, None]
    xmax = jnp.max(jnp.where(mcol > 0.0, gated, -jnp.inf), axis=0)
    xsum = jnp.sum(gated * mcol, axis=0)
    return xmax, xsum / KTOP


def _tc_readout(sagp, dinv, ps, bsc, h):
    def body(sagp_ref, d_ref, ps_ref, bsc_ref, h_ref, out_ref):
        xmax, xmean = _readout_core(sagp_ref, d_ref, ps_ref, bsc_ref, h_ref)
        out_ref[0, :] = xmax
        out_ref[1, :] = xmean

    return pl.pallas_call(
        body,
        out_shape=jax.ShapeDtypeStruct((2, NH), jnp.float32),
    )(sagp, dinv, ps, bsc, h)


def _tc_readout_head(sagp, dinv, ps, bsc, h, x1p, lin1w, lin1b, lin3w, lin3b):
    def body(sagp_ref, d_ref, ps_ref, bsc_ref, h_ref, x1_ref,
             w1_ref, b1_ref, w3_ref, b3_ref, out_ref):
        xmax, xmean = _readout_core(sagp_ref, d_ref, ps_ref, bsc_ref, h_ref)
        z = jnp.concatenate([x1_ref[0], x1_ref[1], xmax, xmean], axis=0)
        zm = jnp.sum(z[:, None] * w1_ref[...], axis=0) + b1_ref[...]
        zm = jnp.maximum(zm, 0.0)
        o = jnp.sum(zm[:, None] * w3_ref[...], axis=0) + b3_ref[...]
        m = jnp.max(o)
        e = o - m
        out_ref[...] = (e - jnp.log(jnp.sum(jnp.exp(e))))[None, :]

    return pl.pallas_call(
        body,
        out_shape=jax.ShapeDtypeStruct((1, NCLS), jnp.float32),
    )(sagp, dinv, ps, bsc, h, x1p, lin1w, lin1b, lin3w, lin3b)


def _rechunk(er):
    return er.reshape(EPAD // RCHUNK, RCHUNK)


def _pad_edges(ei):
    pad = EPAD - E
    pad_dst = N + jnp.arange(pad, dtype=jnp.int32) % (NROW - N)
    srcp = jnp.concatenate(
        [ei[0], jnp.zeros((pad,), jnp.int32)]).reshape(ECH, CHUNK)
    dstp = jnp.concatenate([ei[1], pad_dst]).reshape(ECH, CHUNK)
    return srcp, dstp


def kernel(x, s, edge1_index, edge2_index, batch, W1, b1, Wsc1, bsc1,
           W2, b2, Wsc2, bsc2, lin1_W, lin1_b, lin3_W, lin3_b):
    src1r, dst1r = _pad_edges(edge1_index)
    src2r, dst2r = _pad_edges(edge2_index)

    degp = _sc_degrees(dst1r, dst2r)
    dinv1, dinv2, hs1 = _tc_prep(degp, x, W1)
    hs2 = _tc_big_matmul(s, W2, dinv2.reshape(NROW, 1))

    aggp1 = _sc_row_agg(hs1, _rechunk(src1r), _rechunk(dst1r))
    h1, ps1 = _tc_combine(aggp1, hs1, dinv1, b1, Wsc1.reshape(1, NH))
    sagp1 = _sc_scalar_agg(ps1, src1r, dst1r)
    x1p = _tc_readout(sagp1, dinv1, ps1, bsc1, h1)

    aggp2 = _sc_row_agg(hs2, _rechunk(src2r), _rechunk(dst2r))
    h2, ps2 = _tc_combine(aggp2, hs2, dinv2, b2, Wsc2.reshape(1, NH))
    sagp2 = _sc_scalar_agg(ps2, src2r, dst2r)
    return _tc_readout_head(sagp2, dinv2, ps2, bsc2, h2, x1p,
                            lin1_W, lin1_b, lin3_W, lin3_b)
